# Initial kernel scaffold; baseline (speedup 1.0000x reference)
import jax
import jax.numpy as jnp
from jax.experimental import pallas as pl


def kernel(user_x, feat_embed, has_edge_attr, params, has_edge_index, social_edge_index):
    raise NotImplementedError("write your pallas kernel here")


# speedup vs baseline: 39.2665x; 39.2665x over previous
import functools

import jax
import jax.numpy as jnp
from jax import lax
from jax.experimental import pallas as pl
from jax.experimental.pallas import tpu as pltpu
from jax.experimental.pallas import tpu_sc as plsc

NU = 10000
NF = 256
HID = 64
HEADS = 2
NR = 10112
PAD_E = 163840
K = 256
KD = 2048
NTILE = 16
NW = 2 * NTILE
RPT = NR // NTILE
DSL_R = 320
_F32 = jnp.float32
_I32 = jnp.int32

_MESH = dict(core_axis_name="c", subcore_axis_name="s")


def _leaky(x):
    return jnp.where(x >= 0, x, x * jnp.float32(0.2))


@functools.lru_cache(maxsize=None)
def _make_wgs(nt, edge_split):
    epc = PAD_E // 2 if edge_split else PAD_E
    per_tile = epc // NTILE
    nch = per_tile // K

    @functools.partial(
        pl.kernel,
        out_type=[jax.ShapeDtypeStruct((2 * NR,), _F32),
                  jax.ShapeDtypeStruct((2 * NR, HID), _F32)],
        mesh=plsc.VectorSubcoreMesh(**_MESH),
        compiler_params=pltpu.CompilerParams(needs_layout_passes=False, use_tc_tiling_on_sc=False),
        scratch_types=[
            pltpu.VMEM((NR,), _F32),
            pltpu.VMEM((NR,), _F32),
            pltpu.VMEM((K,), _I32),
            pltpu.VMEM((K,), _I32),
            pltpu.VMEM((K,), _F32),
            pltpu.VMEM((K, HID), _F32),
            pltpu.VMEM_SHARED((NR,), _F32),
            pltpu.VMEM_SHARED((NR, HID), _F32),
            pltpu.SemaphoreType.DMA,
        ],
    )
    def wgs(table_hbm, sS_hbm, sD_hbm, src_hbm, dst_hbm, zs_hbm, za_hbm,
            S_out, ACC_out,
            sS_loc, sD_loc, src_buf, dst_buf, w_buf, rows, s_sh, acc_sh, sem):
        cid = lax.axis_index("c")
        tid = lax.axis_index("s")
        tsel = cid if nt == 2 else 0

        @pl.when(tid == 0)
        def _zero():
            pltpu.sync_copy(zs_hbm, s_sh)
            pltpu.sync_copy(za_hbm, acc_sh)

        pltpu.sync_copy(sS_hbm.at[tsel], sS_loc)
        pltpu.sync_copy(sD_hbm.at[tsel], sD_loc)
        plsc.subcore_barrier()

        ebase = (cid * epc if edge_split else 0) + tid * per_tile
        toff = tsel * NU

        def chunk(ch, carry):
            off = ebase + ch * K
            pltpu.sync_copy(src_hbm.at[pl.ds(off, K)], src_buf)
            pltpu.sync_copy(dst_hbm.at[pl.ds(off, K)], dst_buf)

            def grp(j, c2):
                sv = src_buf[pl.ds(j * 16, 16)]
                dv = dst_buf[pl.ds(j * 16, 16)]
                ss = plsc.load_gather(sS_loc, [sv])
                sd = plsc.load_gather(sD_loc, [dv])
                w_buf[pl.ds(j * 16, 16)] = jnp.exp(_leaky(ss + sd))
                src_buf[pl.ds(j * 16, 16)] = sv + toff
                return c2

            lax.fori_loop(0, K // 16, grp, 0)
            pltpu.sync_copy(w_buf, s_sh.at[dst_buf], add=True)
            pltpu.async_copy(table_hbm.at[src_buf], rows, sem).wait()

            def edge(e, c2):
                lanes = jnp.full((16,), 0, _I32) + e
                wsp = plsc.load_gather(w_buf, [lanes])
                for q in range(HID // 16):
                    rows[e, pl.ds(q * 16, 16)] = rows[e, pl.ds(q * 16, 16)] * wsp
                return c2

            lax.fori_loop(0, K, edge, 0)
            pltpu.sync_copy(rows, acc_sh.at[dst_buf], add=True)
            return carry

        lax.fori_loop(0, nch, chunk, 0)
        plsc.subcore_barrier()
        r0 = tid * RPT
        pltpu.sync_copy(s_sh.at[pl.ds(r0, RPT)],
                        S_out.at[pl.ds(cid * NR + r0, RPT)])
        pltpu.sync_copy(acc_sh.at[pl.ds(r0, RPT)],
                        ACC_out.at[pl.ds(cid * NR + r0, RPT)])

    return wgs


@functools.lru_cache(maxsize=None)
def _make_rel():
    @functools.partial(
        pl.kernel,
        out_type=[jax.ShapeDtypeStruct((12 * NR,), _F32)],
        mesh=plsc.VectorSubcoreMesh(**_MESH),
        compiler_params=pltpu.CompilerParams(needs_layout_passes=False, use_tc_tiling_on_sc=False),
        scratch_types=[
            pltpu.VMEM((NR,), _F32),
            pltpu.VMEM((NR,), _F32),
            pltpu.VMEM((NF,), _F32),
            pltpu.VMEM((NF,), _F32),
            pltpu.VMEM((64,), _F32),
            pltpu.VMEM((K,), _I32),
            pltpu.VMEM((K,), _I32),
            pltpu.VMEM((K,), _F32),
            pltpu.VMEM((K,), _F32),
            [pltpu.VMEM((K,), _F32) for _ in range(6)],
            [pltpu.VMEM_SHARED((NR,), _F32) for _ in range(6)],
        ],
    )
    def rel(sDu_hbm, sSf_hbm, wedge_hbm, u_hbm, f_hbm, om_hbm, z_hbm,
            zs_hbm, out,
            sD0, sD1, sS0, sS1, wg, u_buf, f_buf, om_buf, z_buf, qb, qsh):
        cid = lax.axis_index("c")
        tid = lax.axis_index("s")

        @pl.when(tid == 0)
        def _zero():
            for q in range(6):
                pltpu.sync_copy(zs_hbm, qsh[q])

        pltpu.sync_copy(sDu_hbm.at[0], sD0)
        pltpu.sync_copy(sDu_hbm.at[1], sD1)
        pltpu.sync_copy(sSf_hbm.at[0], sS0)
        pltpu.sync_copy(sSf_hbm.at[1], sS1)
        pltpu.sync_copy(wedge_hbm, wg)
        plsc.subcore_barrier()

        per_tile = (PAD_E // 2) // NTILE
        ebase = cid * (PAD_E // 2) + tid * per_tile
        we00 = wg[pl.ds(0, 16)]
        we01 = wg[pl.ds(16, 16)]
        we10 = wg[pl.ds(32, 16)]
        we11 = wg[pl.ds(48, 16)]
        one = jnp.full((16,), 1.0, _F32)

        def chunk(ch, carry):
            off = ebase + ch * K
            pltpu.sync_copy(u_hbm.at[pl.ds(off, K)], u_buf)
            pltpu.sync_copy(f_hbm.at[pl.ds(off, K)], f_buf)
            pltpu.sync_copy(om_hbm.at[pl.ds(off, K)], om_buf)
            pltpu.sync_copy(z_hbm.at[pl.ds(off, K)], z_buf)

            def grp(j, c2):
                sl = pl.ds(j * 16, 16)
                uv = u_buf[sl]
                fv = f_buf[sl]
                om = om_buf[sl]
                zv = z_buf[sl]
                w0 = jnp.exp(_leaky(plsc.load_gather(sD0, [uv])
                                    + plsc.load_gather(sS0, [fv])))
                w1 = jnp.exp(_leaky(plsc.load_gather(sD1, [uv])
                                    + plsc.load_gather(sS1, [fv])))
                ew0 = one / (one + jnp.exp(-(om * we00 + zv * we01)))
                ew1 = one / (one + jnp.exp(-(om * we10 + zv * we11)))
                c0 = w0 * ew0 * om
                c1 = w1 * ew1 * om
                qb[0][sl] = w0
                qb[1][sl] = w1
                qb[2][sl] = c0
                qb[3][sl] = c1
                qb[4][sl] = c0 * zv
                qb[5][sl] = c1 * zv
                return c2

            lax.fori_loop(0, K // 16, grp, 0)
            for q in range(6):
                pltpu.sync_copy(qb[q], qsh[q].at[u_buf], add=True)
            return carry

        lax.fori_loop(0, per_tile // K, chunk, 0)
        plsc.subcore_barrier()
        r0 = tid * RPT
        for q in range(6):
            pltpu.sync_copy(qsh[q].at[pl.ds(r0, RPT)],
                            out.at[pl.ds((cid * 6 + q) * NR + r0, RPT)])

    return rel


@functools.lru_cache(maxsize=None)
def _make_dsl():
    @functools.partial(
        pl.kernel,
        out_type=[jax.ShapeDtypeStruct((NW * DSL_R,), _F32),
                  jax.ShapeDtypeStruct((NW * DSL_R,), _F32),
                  jax.ShapeDtypeStruct((NW * DSL_R,), _F32)],
        mesh=plsc.VectorSubcoreMesh(**_MESH),
        compiler_params=pltpu.CompilerParams(needs_layout_passes=False, use_tc_tiling_on_sc=False),
        scratch_types=[
            pltpu.VMEM((DSL_R * NF,), _F32),
            pltpu.VMEM((KD,), _I32),
            pltpu.VMEM((KD,), _I32),
            pltpu.VMEM((KD,), _F32),
            pltpu.VMEM((DSL_R, 16), _F32),
            pltpu.VMEM((DSL_R,), _F32),
            pltpu.VMEM((DSL_R,), _F32),
            pltpu.VMEM((DSL_R,), _F32),
        ],
    )
    def dsl(u_hbm, f_hbm, om_hbm, sd_out, t0_out, t255_out,
            shard, u_buf, f_buf, om_buf, spart, srow, t0_loc, t255_loc):
        cid = lax.axis_index("c")
        tid = lax.axis_index("s")
        w = cid * NTILE + tid
        base = w * DSL_R
        zeros16 = jnp.zeros((16,), _F32)

        def z(i, c):
            shard[pl.ds(i * 16, 16)] = zeros16
            return c

        lax.fori_loop(0, DSL_R * NF // 16, z, 0)

        def chunk(ch, carry):
            off = ch * KD
            pltpu.sync_copy(u_hbm.at[pl.ds(off, KD)], u_buf)
            pltpu.sync_copy(f_hbm.at[pl.ds(off, KD)], f_buf)
            pltpu.sync_copy(om_hbm.at[pl.ds(off, KD)], om_buf)

            def grp(j, c2):
                sl = pl.ds(j * 16, 16)
                uv = u_buf[sl]
                fv = f_buf[sl]
                m = jnp.logical_and(uv >= base, uv < base + DSL_R)
                lidx = (uv - base) * NF + fv
                lidx = jnp.where(m, lidx, 0)
                val = jnp.full((16,), 1.0, _F32) - om_buf[sl]
                plsc.store_scatter(shard, [lidx], val, mask=m)
                return c2

            lax.fori_loop(0, KD // 16, grp, 0)
            return carry

        lax.fori_loop(0, PAD_E // KD, chunk, 0)

        def rowsum(r, carry):
            def qs(q, acc):
                return acc + shard[pl.ds(r * NF + q * 16, 16)]

            spart[r, pl.ds(0, 16)] = lax.fori_loop(0, NF // 16, qs, zeros16)
            return carry

        lax.fori_loop(0, DSL_R, rowsum, 0)

        iota = jnp.arange(16, dtype=_I32)

        def rowred(rg, carry):
            ridx = iota + rg * 16
            tot = jnp.zeros((16,), _F32)
            for c in range(16):
                cidx = jnp.full((16,), c, _I32)
                tot = tot + plsc.load_gather(spart, [ridx, cidx])
            srow[pl.ds(rg * 16, 16)] = tot
            return carry

        lax.fori_loop(0, DSL_R // 16, rowred, 0)

        def cols(rg, carry):
            ridx = (iota + rg * 16) * NF
            t0_loc[pl.ds(rg * 16, 16)] = plsc.load_gather(shard, [ridx])
            t255_loc[pl.ds(rg * 16, 16)] = plsc.load_gather(
                shard, [ridx + (NF - 1)])
            return carry

        lax.fori_loop(0, DSL_R // 16, cols, 0)
        pltpu.sync_copy(srow, sd_out.at[pl.ds(base, DSL_R)])
        pltpu.sync_copy(t0_loc, t0_out.at[pl.ds(base, DSL_R)])
        pltpu.sync_copy(t255_loc, t255_out.at[pl.ds(base, DSL_R)])

    return dsl


def _rel_edge_pass(*args):
    return _make_rel()(*args)


def _wgs_heads(*args):
    return _make_wgs(2, False)(*args)


def _wgs_split(*args):
    return _make_wgs(1, True)(*args)


def _dsl_pass(*args):
    return _make_dsl()(*args)


_BLK = 1000


def _tc_pre_feat(feat_embed, p_Wf, p_bf, p_Wsrc, p_a, o_ssF):
    xf = feat_embed[...] @ p_Wf[...].T + p_bf[...][None, :]
    hsF = xf @ p_Wsrc[...].T
    a = p_a[...][0]
    hsF3 = hsF.reshape(NF, HEADS, HID)
    for h in range(HEADS):
        o_ssF[h, :] = hsF3[:, h, :] @ a[h, HID:]


def _tc_pre_user(user_x, p_Wu, p_bu, p_Wdst, p_a, p_Wsoc, p_as, p_ad,
                 o_hdU, o_sdU, o_socT, o_socS, o_socD):
    xu = user_x[...] @ p_Wu[...].T + p_bu[...][None, :]
    hdU = xu @ p_Wdst[...].T
    o_hdU[...] = hdU
    a = p_a[...][0]
    hdU3 = hdU.reshape(_BLK, HEADS, HID)
    for h in range(HEADS):
        o_sdU[:, h] = hdU3[:, h, :] @ a[h, :HID]
    hsoc = xu @ p_Wsoc[...].T
    hsoc3 = hsoc.reshape(_BLK, HEADS, HID)
    for h in range(HEADS):
        o_socT[h, :, :] = hsoc3[:, h, :]
        o_socS[:, h] = hsoc3[:, h, :] @ p_as[...][0, h]
        o_socD[:, h] = hsoc3[:, h, :] @ p_ad[...][0, h]


def _tc_rel(hdU, rel6, p_Wz, p_rb, o_rel):
    r = rel6[...]
    hdUv = hdU[...]
    Wzv = p_Wz[...][:, 0]
    rel_out = jnp.zeros((NU, HID), _F32)
    for h in range(HEADS):
        S = r[h, :NU] + r[6 + h, :NU]
        A = r[2 + h, :NU] + r[8 + h, :NU]
        B = r[4 + h, :NU] + r[10 + h, :NU]
        inv = 1.0 / (S + 1e-16)
        rel_out = rel_out + (A * inv)[:, None] * hdUv[:, h * HID:(h + 1) * HID] \
            + (B * inv)[:, None] * Wzv[None, :]
    o_rel[...] = rel_out / HEADS + p_rb[...][None, :]


def _tc_mid(rel_out, socT, socS, socD, Ssoc, ACCsoc,
            p_sb, p_g1w, p_g1as, p_g1ad,
            o_user, o_h1, o_s1S, o_s1D):
    Ss = Ssoc[...]
    ACC = ACCsoc[...]
    soc_out = jnp.zeros((NU, HID), _F32)
    for h in range(HEADS):
        hsoc = socT[...][h * NU:(h + 1) * NU, :]
        wself = jnp.exp(_leaky(socS[...][h, :NU] + socD[...][h, :NU]))
        S = Ss[h, :NU] + wself
        AC = ACC[h, :NU, :] + wself[:, None] * hsoc
        soc_out = soc_out + AC / (S + 1e-16)[:, None]
    soc_out = soc_out / HEADS + p_sb[...][None, :]

    user = jnp.maximum(rel_out[...] + soc_out, 0.0)
    o_user[...] = user
    h1 = user @ p_g1w[...].T
    o_h1[...] = h1
    o_s1S[0, :] = h1 @ p_g1as[...][0, 0]
    o_s1D[0, :] = h1 @ p_g1ad[...][0, 0]


def _tc_g1fin(h1, s1S, s1D, Sg, ACCg, p_g1b, p_g2w, p_g2as, p_g2ad,
              o_h2, o_s2S, o_s2D):
    S2 = Sg[...]
    A2 = ACCg[...]
    wself = jnp.exp(_leaky(s1S[...][0, :] + s1D[...][0, :]))
    S = S2[0, :NU] + S2[1, :NU] + wself
    AC = A2[0, :NU, :] + A2[1, :NU, :] + wself[:, None] * h1[...]
    hr = jnp.maximum(AC / (S + 1e-16)[:, None] + p_g1b[...][None, :], 0.0)
    h2 = hr @ p_g2w[...].T
    o_h2[...] = h2
    o_s2S[0, :] = h2 @ p_g2as[...][0, 0]
    o_s2D[0, :] = h2 @ p_g2ad[...][0, 0]


def _tc_att(user, h2, s2S, s2D, Sg, ACCg,
            p_g2b, p_Wq, p_bq, p_Wk, p_bk, p_Wv, p_bv, p_Wne, p_bne,
            o_g):
    S2 = Sg[...]
    A2 = ACCg[...]
    wself = jnp.exp(_leaky(s2S[...][0, :] + s2D[...][0, :]))
    S = S2[0, :NU] + S2[1, :NU] + wself
    AC = A2[0, :NU, :] + A2[1, :NU, :] + wself[:, None] * h2[...]
    neighbor_h = AC / (S + 1e-16)[:, None] + p_g2b[...][None, :]

    u = user[...]
    Q = u @ p_Wq[...].T + p_bq[...][None, :]
    Km = neighbor_h @ p_Wk[...].T + p_bk[...][None, :]
    V = neighbor_h @ p_Wv[...].T + p_bv[...][None, :]
    scores = (Q * Km).sum(-1) / jnp.sqrt(jnp.float32(HID))
    wat = jax.nn.softmax(scores, axis=0)
    gv = wat @ V
    o_g[0, :] = gv @ p_Wne[...].T + p_bne[...]


def _tc_out(user, gin, sda, t0a, t255a,
            p_Wme, p_bme, p_cw, p_cb, p_dw, p_db, p_o1w, p_o1b,
            p_o2w, p_o2b,
            o_o, o_d):
    g = gin[...][0]
    user_h = user[...] @ p_Wme[...].T + p_bme[...][None, :] + g[None, :]

    Sd = sda[...]
    t0 = t0a[...]
    t255 = t255a[...]
    cw = p_cw[...]
    dw = p_dw[...]
    v1 = dw @ cw.sum(1) / NF
    v2 = dw @ cw[:, 0] / NF
    v3 = dw @ cw[:, 2] / NF
    v0 = dw @ p_cb[...] + p_db[...]
    d = (Sd[:, None] * v1[None, :] - t255[:, None] * v2[None, :]
         - t0[:, None] * v3[None, :] + v0[None, :])
    o_d[...] = d

    W1 = p_o1w[...]
    gterm = g @ W1[:, HID:2 * HID].T + p_o1b[...]
    o1 = jnp.maximum(user_h @ W1[:, :HID].T + d @ W1[:, 2 * HID:].T
                     + gterm[None, :], 0.0)
    o_o[...] = o1 @ p_o2w[...].T + p_o2b[...][None, :]


def _pallas(body, **kw):
    return pl.pallas_call(body, **kw)


def _tc_call(body, out_shapes, *args, name):
    return _pallas(
        body,
        out_shape=[jax.ShapeDtypeStruct(s, _F32) for s in out_shapes],
        name=name,
    )(*args)


def kernel(user_x, feat_embed, has_edge_attr, params, has_edge_index,
           social_edge_index):
    p = params
    E = has_edge_index.shape[1]
    npad = PAD_E - E
    trash = (NU + (jnp.arange(npad, dtype=_I32) % 64)).astype(_I32)
    zi = jnp.zeros((npad,), _I32)
    zf = jnp.zeros((npad,), _F32)

    u_e = jnp.concatenate([has_edge_index[0].astype(_I32), trash])
    f_e = jnp.concatenate([has_edge_index[1].astype(_I32), zi])
    om_e = jnp.concatenate([has_edge_attr[:, 0], zf])
    z_e = jnp.concatenate([has_edge_attr[:, 1], zf])
    s_src = jnp.concatenate([social_edge_index[0].astype(_I32), zi])
    s_dst = jnp.concatenate([social_edge_index[1].astype(_I32), trash])

    zsN = jnp.zeros((NR,), _F32)
    zsA = jnp.zeros((NR, HID), _F32)
    wedge = jnp.repeat(p['rel_Wedge'].reshape(4), 16)

    ssF, = _tc_call(
        _tc_pre_feat, [(HEADS, NF)],
        feat_embed, p['feat_proj_w'], p['feat_proj_b'], p['rel_Wsrc'],
        p['rel_a'], name="tc_pre_feat")

    full = lambda shp: pl.BlockSpec(shp, lambda i: tuple(0 for _ in shp))
    hdU, sdU, socT3, socS, socD = _pallas(
        _tc_pre_user,
        grid=(NU // _BLK,),
        in_specs=[
            pl.BlockSpec((_BLK, 128), lambda i: (i, 0)),
            full((HID, 128)), full((HID,)), full((HEADS * HID, HID)),
            full((1, HEADS, 2 * HID)), full((HEADS * HID, HID)),
            full((1, HEADS, HID)), full((1, HEADS, HID)),
        ],
        out_specs=[
            pl.BlockSpec((_BLK, HEADS * HID), lambda i: (i, 0)),
            pl.BlockSpec((_BLK, HEADS), lambda i: (i, 0)),
            pl.BlockSpec((HEADS, _BLK, HID), lambda i: (0, i, 0)),
            pl.BlockSpec((_BLK, HEADS), lambda i: (i, 0)),
            pl.BlockSpec((_BLK, HEADS), lambda i: (i, 0)),
        ],
        out_shape=[
            jax.ShapeDtypeStruct((NU, HEADS * HID), _F32),
            jax.ShapeDtypeStruct((NU, HEADS), _F32),
            jax.ShapeDtypeStruct((HEADS, NU, HID), _F32),
            jax.ShapeDtypeStruct((NU, HEADS), _F32),
            jax.ShapeDtypeStruct((NU, HEADS), _F32),
        ],
        name="tc_pre_user",
    )(user_x, p['user_proj_w'], p['user_proj_b'], p['rel_Wdst'], p['rel_a'],
      p['soc_w'], p['soc_as'], p['soc_ad'])

    padn = lambda x: jnp.pad(x, ((0, 0), (0, NR - NU)))
    socT = socT3.reshape(HEADS * NU, HID)
    sdU = sdU.T
    socS = socS.T
    socD = socD.T

    rel6, = _rel_edge_pass(padn(sdU), ssF, wedge, u_e, f_e, om_e, z_e, zsN)
    rel6 = rel6.reshape(12, NR)
    Ssoc, ACCsoc = _wgs_heads(socT, padn(socS), padn(socD), s_src, s_dst,
                              zsN, zsA)
    Ssoc = Ssoc.reshape(2, NR)
    ACCsoc = ACCsoc.reshape(2, NR, HID)
    sda, t0a, t255a = _dsl_pass(u_e, f_e, om_e)
    sda = sda[:NU]
    t0a = t0a[:NU]
    t255a = t255a[:NU]

    rel_out, = _tc_call(
        _tc_rel, [(NU, HID)],
        hdU, rel6, p['rel_Wz'], p['rel_bias'], name="tc_rel")

    user, h1, s1S, s1D = _tc_call(
        _tc_mid,
        [(NU, HID), (NU, HID), (1, NU), (1, NU)],
        rel_out, socT, padn(socS), padn(socD), Ssoc, ACCsoc,
        p['soc_b'], p['g1_w'], p['g1_as'], p['g1_ad'],
        name="tc_mid")

    Sg1, ACC1 = _wgs_split(h1, padn(s1S), padn(s1D), s_src, s_dst, zsN, zsA)
    Sg1 = Sg1.reshape(2, NR)
    ACC1 = ACC1.reshape(2, NR, HID)

    h2, s2S, s2D = _tc_call(
        _tc_g1fin,
        [(NU, HID), (1, NU), (1, NU)],
        h1, s1S, s1D, Sg1, ACC1, p['g1_b'], p['g2_w'], p['g2_as'], p['g2_ad'],
        name="tc_g1fin")

    Sg2, ACC2 = _wgs_split(h2, padn(s2S), padn(s2D), s_src, s_dst, zsN, zsA)
    Sg2 = Sg2.reshape(2, NR)
    ACC2 = ACC2.reshape(2, NR, HID)

    g2d, = _tc_call(
        _tc_att, [(1, HID)],
        user, h2, s2S, s2D, Sg2, ACC2,
        p['g2_b'], p['Wq'], p['bq'], p['Wk'], p['bk'], p['Wv'], p['bv'],
        p['Wne'], p['bne'],
        name="tc_att")

    o, d = _tc_call(
        _tc_out, [(NU, HID), (NU, HID)],
        user, g2d, sda, t0a, t255a,
        p['Wme'], p['bme'],
        p['conv_w'][:, 0, :], p['conv_b'], p['dsl_w'], p['dsl_b'],
        p['op1_w'], p['op1_b'], p['op2_w'], p['op2_b'],
        name="tc_out")

    return o, g2d[0], d


# speedup vs baseline: 48.3673x; 1.2318x over previous
import functools

import jax
import jax.numpy as jnp
from jax import lax
from jax.experimental import pallas as pl
from jax.experimental.pallas import tpu as pltpu
from jax.experimental.pallas import tpu_sc as plsc

NU = 10000
NF = 256
HID = 64
HEADS = 2
NR = 10112
PAD_E = 163840
K = 1024
KD = 8192
NTILE = 16
NW = 2 * NTILE
RPT = NR // NTILE
DSL_R = 320
_F32 = jnp.float32
_I32 = jnp.int32

_MESH = dict(core_axis_name="c", subcore_axis_name="s")


def _leaky(x):
    return jnp.where(x >= 0, x, x * jnp.float32(0.2))


@functools.lru_cache(maxsize=None)
def _make_wgs(nt, edge_split):
    epc = PAD_E // 2 if edge_split else PAD_E
    per_tile = epc // NTILE
    nch = per_tile // K

    @functools.partial(
        pl.kernel,
        out_type=[jax.ShapeDtypeStruct((2 * NR,), _F32),
                  jax.ShapeDtypeStruct((2 * NR, HID), _F32)],
        mesh=plsc.VectorSubcoreMesh(**_MESH),
        compiler_params=pltpu.CompilerParams(needs_layout_passes=False, use_tc_tiling_on_sc=False),
        scratch_types=[
            pltpu.VMEM((NR,), _F32),
            pltpu.VMEM((NR,), _F32),
            pltpu.VMEM((K,), _I32),
            pltpu.VMEM((K,), _I32),
            pltpu.VMEM((K,), _I32),
            pltpu.VMEM((K,), _F32),
            pltpu.VMEM((K, HID), _F32),
            pltpu.VMEM_SHARED((NR,), _F32),
            pltpu.VMEM_SHARED((NR, HID), _F32),
            pltpu.SemaphoreType.DMA,
            pltpu.SemaphoreType.DMA,
        ],
    )
    def wgs(table_hbm, sS_hbm, sD_hbm, src_hbm, dst_hbm, zs_hbm, za_hbm,
            S_out, ACC_out,
            sS_loc, sD_loc, src_buf, dst_buf, gidx, w_buf, rows, s_sh, acc_sh,
            sem, sem2):
        cid = lax.axis_index("c")
        tid = lax.axis_index("s")
        tsel = cid if nt == 2 else 0

        @pl.when(tid == 0)
        def _zero():
            pltpu.sync_copy(zs_hbm, s_sh)
            pltpu.sync_copy(za_hbm, acc_sh)

        pltpu.sync_copy(sS_hbm.at[tsel], sS_loc)
        pltpu.sync_copy(sD_hbm.at[tsel], sD_loc)
        plsc.subcore_barrier()

        ebase = (cid * epc if edge_split else 0) + tid * per_tile
        toff = tsel * NU

        def chunk(ch, carry):
            off = ebase + ch * K
            pltpu.sync_copy(src_hbm.at[pl.ds(off, K)], src_buf)
            pltpu.sync_copy(dst_hbm.at[pl.ds(off, K)], dst_buf)

            def ofs(j, c2):
                gidx[pl.ds(j * 16, 16)] = src_buf[pl.ds(j * 16, 16)] + toff
                return c2

            lax.fori_loop(0, K // 16, ofs, 0)
            gdma = pltpu.async_copy(table_hbm.at[gidx], rows, sem)

            def grp(j, c2):
                sv = src_buf[pl.ds(j * 16, 16)]
                dv = dst_buf[pl.ds(j * 16, 16)]
                ss = plsc.load_gather(sS_loc, [sv])
                sd = plsc.load_gather(sD_loc, [dv])
                w_buf[pl.ds(j * 16, 16)] = jnp.exp(_leaky(ss + sd))
                return c2

            lax.fori_loop(0, K // 16, grp, 0)
            wdma = pltpu.async_copy(w_buf, s_sh.at[dst_buf], sem2, add=True)
            gdma.wait()

            def edge(e, c2):
                lanes = jnp.full((16,), 0, _I32) + e
                wsp = plsc.load_gather(w_buf, [lanes])
                for q in range(HID // 16):
                    rows[e, pl.ds(q * 16, 16)] = rows[e, pl.ds(q * 16, 16)] * wsp
                return c2

            lax.fori_loop(0, K, edge, 0)
            pltpu.sync_copy(rows, acc_sh.at[dst_buf], add=True)
            wdma.wait()
            return carry

        lax.fori_loop(0, nch, chunk, 0)
        plsc.subcore_barrier()
        r0 = tid * RPT
        pltpu.sync_copy(s_sh.at[pl.ds(r0, RPT)],
                        S_out.at[pl.ds(cid * NR + r0, RPT)])
        pltpu.sync_copy(acc_sh.at[pl.ds(r0, RPT)],
                        ACC_out.at[pl.ds(cid * NR + r0, RPT)])

    return wgs


@functools.lru_cache(maxsize=None)
def _make_rel():
    @functools.partial(
        pl.kernel,
        out_type=[jax.ShapeDtypeStruct((12 * NR,), _F32)],
        mesh=plsc.VectorSubcoreMesh(**_MESH),
        compiler_params=pltpu.CompilerParams(needs_layout_passes=False, use_tc_tiling_on_sc=False),
        scratch_types=[
            pltpu.VMEM((NR,), _F32),
            pltpu.VMEM((NR,), _F32),
            pltpu.VMEM((NF,), _F32),
            pltpu.VMEM((NF,), _F32),
            pltpu.VMEM((64,), _F32),
            pltpu.VMEM((K,), _I32),
            pltpu.VMEM((K,), _I32),
            pltpu.VMEM((K,), _F32),
            pltpu.VMEM((K,), _F32),
            [pltpu.VMEM((K,), _F32) for _ in range(6)],
            [pltpu.VMEM_SHARED((NR,), _F32) for _ in range(6)],
        ],
    )
    def rel(sDu_hbm, sSf_hbm, wedge_hbm, u_hbm, f_hbm, om_hbm, z_hbm,
            zs_hbm, out,
            sD0, sD1, sS0, sS1, wg, u_buf, f_buf, om_buf, z_buf, qb, qsh):
        cid = lax.axis_index("c")
        tid = lax.axis_index("s")

        @pl.when(tid == 0)
        def _zero():
            for q in range(6):
                pltpu.sync_copy(zs_hbm, qsh[q])

        pltpu.sync_copy(sDu_hbm.at[0], sD0)
        pltpu.sync_copy(sDu_hbm.at[1], sD1)
        pltpu.sync_copy(sSf_hbm.at[0], sS0)
        pltpu.sync_copy(sSf_hbm.at[1], sS1)
        pltpu.sync_copy(wedge_hbm, wg)
        plsc.subcore_barrier()

        per_tile = (PAD_E // 2) // NTILE
        ebase = cid * (PAD_E // 2) + tid * per_tile
        we00 = wg[pl.ds(0, 16)]
        we01 = wg[pl.ds(16, 16)]
        we10 = wg[pl.ds(32, 16)]
        we11 = wg[pl.ds(48, 16)]
        one = jnp.full((16,), 1.0, _F32)

        def chunk(ch, carry):
            off = ebase + ch * K
            pltpu.sync_copy(u_hbm.at[pl.ds(off, K)], u_buf)
            pltpu.sync_copy(f_hbm.at[pl.ds(off, K)], f_buf)
            pltpu.sync_copy(om_hbm.at[pl.ds(off, K)], om_buf)
            pltpu.sync_copy(z_hbm.at[pl.ds(off, K)], z_buf)

            def grp(j, c2):
                sl = pl.ds(j * 16, 16)
                uv = u_buf[sl]
                fv = f_buf[sl]
                om = om_buf[sl]
                zv = z_buf[sl]
                w0 = jnp.exp(_leaky(plsc.load_gather(sD0, [uv])
                                    + plsc.load_gather(sS0, [fv])))
                w1 = jnp.exp(_leaky(plsc.load_gather(sD1, [uv])
                                    + plsc.load_gather(sS1, [fv])))
                ew0 = one / (one + jnp.exp(-(om * we00 + zv * we01)))
                ew1 = one / (one + jnp.exp(-(om * we10 + zv * we11)))
                c0 = w0 * ew0 * om
                c1 = w1 * ew1 * om
                qb[0][sl] = w0
                qb[1][sl] = w1
                qb[2][sl] = c0
                qb[3][sl] = c1
                qb[4][sl] = c0 * zv
                qb[5][sl] = c1 * zv
                return c2

            lax.fori_loop(0, K // 16, grp, 0)
            for q in range(6):
                pltpu.sync_copy(qb[q], qsh[q].at[u_buf], add=True)
            return carry

        lax.fori_loop(0, per_tile // K, chunk, 0)
        plsc.subcore_barrier()
        r0 = tid * RPT
        for q in range(6):
            pltpu.sync_copy(qsh[q].at[pl.ds(r0, RPT)],
                            out.at[pl.ds((cid * 6 + q) * NR + r0, RPT)])

    return rel


@functools.lru_cache(maxsize=None)
def _make_dsl():
    @functools.partial(
        pl.kernel,
        out_type=[jax.ShapeDtypeStruct((NW * DSL_R,), _F32),
                  jax.ShapeDtypeStruct((NW * DSL_R,), _F32),
                  jax.ShapeDtypeStruct((NW * DSL_R,), _F32)],
        mesh=plsc.VectorSubcoreMesh(**_MESH),
        compiler_params=pltpu.CompilerParams(needs_layout_passes=False, use_tc_tiling_on_sc=False),
        scratch_types=[
            pltpu.VMEM((DSL_R * NF,), _F32),
            pltpu.VMEM((KD,), _I32),
            pltpu.VMEM((KD,), _I32),
            pltpu.VMEM((KD,), _F32),
            pltpu.VMEM((DSL_R, 16), _F32),
            pltpu.VMEM((DSL_R,), _F32),
            pltpu.VMEM((DSL_R,), _F32),
            pltpu.VMEM((DSL_R,), _F32),
        ],
    )
    def dsl(u_hbm, f_hbm, om_hbm, sd_out, t0_out, t255_out,
            shard, u_buf, f_buf, om_buf, spart, srow, t0_loc, t255_loc):
        cid = lax.axis_index("c")
        tid = lax.axis_index("s")
        w = cid * NTILE + tid
        base = w * DSL_R
        zeros16 = jnp.zeros((16,), _F32)

        def z(i, c):
            shard[pl.ds(i * 16, 16)] = zeros16
            return c

        lax.fori_loop(0, DSL_R * NF // 16, z, 0)

        def chunk(ch, carry):
            off = ch * KD
            pltpu.sync_copy(u_hbm.at[pl.ds(off, KD)], u_buf)
            pltpu.sync_copy(f_hbm.at[pl.ds(off, KD)], f_buf)
            pltpu.sync_copy(om_hbm.at[pl.ds(off, KD)], om_buf)

            def grp(j, c2):
                sl = pl.ds(j * 16, 16)
                uv = u_buf[sl]
                fv = f_buf[sl]
                m = jnp.logical_and(uv >= base, uv < base + DSL_R)
                lidx = (uv - base) * NF + fv
                lidx = jnp.where(m, lidx, 0)
                val = jnp.full((16,), 1.0, _F32) - om_buf[sl]
                plsc.store_scatter(shard, [lidx], val, mask=m)
                return c2

            lax.fori_loop(0, KD // 16, grp, 0)
            return carry

        lax.fori_loop(0, PAD_E // KD, chunk, 0)

        def rowsum(r, carry):
            def qs(q, acc):
                return acc + shard[pl.ds(r * NF + q * 16, 16)]

            spart[r, pl.ds(0, 16)] = lax.fori_loop(0, NF // 16, qs, zeros16)
            return carry

        lax.fori_loop(0, DSL_R, rowsum, 0)

        iota = jnp.arange(16, dtype=_I32)

        def rowred(rg, carry):
            ridx = iota + rg * 16
            tot = jnp.zeros((16,), _F32)
            for c in range(16):
                cidx = jnp.full((16,), c, _I32)
                tot = tot + plsc.load_gather(spart, [ridx, cidx])
            srow[pl.ds(rg * 16, 16)] = tot
            return carry

        lax.fori_loop(0, DSL_R // 16, rowred, 0)

        def cols(rg, carry):
            ridx = (iota + rg * 16) * NF
            t0_loc[pl.ds(rg * 16, 16)] = plsc.load_gather(shard, [ridx])
            t255_loc[pl.ds(rg * 16, 16)] = plsc.load_gather(
                shard, [ridx + (NF - 1)])
            return carry

        lax.fori_loop(0, DSL_R // 16, cols, 0)
        pltpu.sync_copy(srow, sd_out.at[pl.ds(base, DSL_R)])
        pltpu.sync_copy(t0_loc, t0_out.at[pl.ds(base, DSL_R)])
        pltpu.sync_copy(t255_loc, t255_out.at[pl.ds(base, DSL_R)])

    return dsl


def _rel_edge_pass(*args):
    return _make_rel()(*args)


def _wgs_heads(*args):
    return _make_wgs(2, False)(*args)


def _wgs_split(*args):
    return _make_wgs(1, True)(*args)


def _dsl_pass(*args):
    return _make_dsl()(*args)


_BLK = 1000


def _tc_pre_feat(feat_embed, p_Wf, p_bf, p_Wsrc, p_a, o_ssF):
    xf = feat_embed[...] @ p_Wf[...].T + p_bf[...][None, :]
    hsF = xf @ p_Wsrc[...].T
    a = p_a[...][0]
    hsF3 = hsF.reshape(NF, HEADS, HID)
    for h in range(HEADS):
        o_ssF[h, :] = hsF3[:, h, :] @ a[h, HID:]


def _tc_pre_user(user_x, p_Wu, p_bu, p_Wdst, p_a, p_Wsoc, p_as, p_ad,
                 o_hdU, o_sdU, o_socT, o_socS, o_socD):
    xu = user_x[...] @ p_Wu[...].T + p_bu[...][None, :]
    hdU = xu @ p_Wdst[...].T
    o_hdU[...] = hdU
    a = p_a[...][0]
    hdU3 = hdU.reshape(_BLK, HEADS, HID)
    for h in range(HEADS):
        o_sdU[:, h] = hdU3[:, h, :] @ a[h, :HID]
    hsoc = xu @ p_Wsoc[...].T
    hsoc3 = hsoc.reshape(_BLK, HEADS, HID)
    for h in range(HEADS):
        o_socT[h, :, :] = hsoc3[:, h, :]
        o_socS[:, h] = hsoc3[:, h, :] @ p_as[...][0, h]
        o_socD[:, h] = hsoc3[:, h, :] @ p_ad[...][0, h]


def _tc_rel(hdU, rel6, p_Wz, p_rb, o_rel):
    r = rel6[...]
    hdUv = hdU[...]
    Wzv = p_Wz[...][:, 0]
    rel_out = jnp.zeros((NU, HID), _F32)
    for h in range(HEADS):
        S = r[h, :NU] + r[6 + h, :NU]
        A = r[2 + h, :NU] + r[8 + h, :NU]
        B = r[4 + h, :NU] + r[10 + h, :NU]
        inv = 1.0 / (S + 1e-16)
        rel_out = rel_out + (A * inv)[:, None] * hdUv[:, h * HID:(h + 1) * HID] \
            + (B * inv)[:, None] * Wzv[None, :]
    o_rel[...] = rel_out / HEADS + p_rb[...][None, :]


def _tc_mid(rel_out, socT, socS, socD, Ssoc, ACCsoc,
            p_sb, p_g1w, p_g1as, p_g1ad,
            o_user, o_h1, o_s1S, o_s1D):
    Ss = Ssoc[...]
    ACC = ACCsoc[...]
    soc_out = jnp.zeros((NU, HID), _F32)
    for h in range(HEADS):
        hsoc = socT[...][h * NU:(h + 1) * NU, :]
        wself = jnp.exp(_leaky(socS[...][h, :NU] + socD[...][h, :NU]))
        S = Ss[h, :NU] + wself
        AC = ACC[h, :NU, :] + wself[:, None] * hsoc
        soc_out = soc_out + AC / (S + 1e-16)[:, None]
    soc_out = soc_out / HEADS + p_sb[...][None, :]

    user = jnp.maximum(rel_out[...] + soc_out, 0.0)
    o_user[...] = user
    h1 = user @ p_g1w[...].T
    o_h1[...] = h1
    o_s1S[0, :] = h1 @ p_g1as[...][0, 0]
    o_s1D[0, :] = h1 @ p_g1ad[...][0, 0]


def _tc_g1fin(h1, s1S, s1D, Sg, ACCg, p_g1b, p_g2w, p_g2as, p_g2ad,
              o_h2, o_s2S, o_s2D):
    S2 = Sg[...]
    A2 = ACCg[...]
    wself = jnp.exp(_leaky(s1S[...][0, :] + s1D[...][0, :]))
    S = S2[0, :NU] + S2[1, :NU] + wself
    AC = A2[0, :NU, :] + A2[1, :NU, :] + wself[:, None] * h1[...]
    hr = jnp.maximum(AC / (S + 1e-16)[:, None] + p_g1b[...][None, :], 0.0)
    h2 = hr @ p_g2w[...].T
    o_h2[...] = h2
    o_s2S[0, :] = h2 @ p_g2as[...][0, 0]
    o_s2D[0, :] = h2 @ p_g2ad[...][0, 0]


def _tc_att(user, h2, s2S, s2D, Sg, ACCg,
            p_g2b, p_Wq, p_bq, p_Wk, p_bk, p_Wv, p_bv, p_Wne, p_bne,
            o_g):
    S2 = Sg[...]
    A2 = ACCg[...]
    wself = jnp.exp(_leaky(s2S[...][0, :] + s2D[...][0, :]))
    S = S2[0, :NU] + S2[1, :NU] + wself
    AC = A2[0, :NU, :] + A2[1, :NU, :] + wself[:, None] * h2[...]
    neighbor_h = AC / (S + 1e-16)[:, None] + p_g2b[...][None, :]

    u = user[...]
    Q = u @ p_Wq[...].T + p_bq[...][None, :]
    Km = neighbor_h @ p_Wk[...].T + p_bk[...][None, :]
    V = neighbor_h @ p_Wv[...].T + p_bv[...][None, :]
    scores = (Q * Km).sum(-1) / jnp.sqrt(jnp.float32(HID))
    wat = jax.nn.softmax(scores, axis=0)
    gv = wat @ V
    o_g[0, :] = gv @ p_Wne[...].T + p_bne[...]


def _tc_out(user, gin, sda, t0a, t255a,
            p_Wme, p_bme, p_cw, p_cb, p_dw, p_db, p_o1w, p_o1b,
            p_o2w, p_o2b,
            o_o, o_d):
    g = gin[...][0]
    user_h = user[...] @ p_Wme[...].T + p_bme[...][None, :] + g[None, :]

    Sd = sda[...]
    t0 = t0a[...]
    t255 = t255a[...]
    cw = p_cw[...]
    dw = p_dw[...]
    v1 = dw @ cw.sum(1) / NF
    v2 = dw @ cw[:, 0] / NF
    v3 = dw @ cw[:, 2] / NF
    v0 = dw @ p_cb[...] + p_db[...]
    d = (Sd[:, None] * v1[None, :] - t255[:, None] * v2[None, :]
         - t0[:, None] * v3[None, :] + v0[None, :])
    o_d[...] = d

    W1 = p_o1w[...]
    gterm = g @ W1[:, HID:2 * HID].T + p_o1b[...]
    o1 = jnp.maximum(user_h @ W1[:, :HID].T + d @ W1[:, 2 * HID:].T
                     + gterm[None, :], 0.0)
    o_o[...] = o1 @ p_o2w[...].T + p_o2b[...][None, :]


def _pallas(body, **kw):
    return pl.pallas_call(body, **kw)


def _tc_call(body, out_shapes, *args, name):
    return _pallas(
        body,
        out_shape=[jax.ShapeDtypeStruct(s, _F32) for s in out_shapes],
        name=name,
    )(*args)


def kernel(user_x, feat_embed, has_edge_attr, params, has_edge_index,
           social_edge_index):
    p = params
    E = has_edge_index.shape[1]
    npad = PAD_E - E
    trash = (NU + (jnp.arange(npad, dtype=_I32) % 64)).astype(_I32)
    zi = jnp.zeros((npad,), _I32)
    zf = jnp.zeros((npad,), _F32)

    u_e = jnp.concatenate([has_edge_index[0].astype(_I32), trash])
    f_e = jnp.concatenate([has_edge_index[1].astype(_I32), zi])
    om_e = jnp.concatenate([has_edge_attr[:, 0], zf])
    z_e = jnp.concatenate([has_edge_attr[:, 1], zf])
    s_src = jnp.concatenate([social_edge_index[0].astype(_I32), zi])
    s_dst = jnp.concatenate([social_edge_index[1].astype(_I32), trash])

    zsN = jnp.zeros((NR,), _F32)
    zsA = jnp.zeros((NR, HID), _F32)
    wedge = jnp.repeat(p['rel_Wedge'].reshape(4), 16)

    ssF, = _tc_call(
        _tc_pre_feat, [(HEADS, NF)],
        feat_embed, p['feat_proj_w'], p['feat_proj_b'], p['rel_Wsrc'],
        p['rel_a'], name="tc_pre_feat")

    full = lambda shp: pl.BlockSpec(shp, lambda i: tuple(0 for _ in shp))
    hdU, sdU, socT3, socS, socD = _pallas(
        _tc_pre_user,
        grid=(NU // _BLK,),
        in_specs=[
            pl.BlockSpec((_BLK, 128), lambda i: (i, 0)),
            full((HID, 128)), full((HID,)), full((HEADS * HID, HID)),
            full((1, HEADS, 2 * HID)), full((HEADS * HID, HID)),
            full((1, HEADS, HID)), full((1, HEADS, HID)),
        ],
        out_specs=[
            pl.BlockSpec((_BLK, HEADS * HID), lambda i: (i, 0)),
            pl.BlockSpec((_BLK, HEADS), lambda i: (i, 0)),
            pl.BlockSpec((HEADS, _BLK, HID), lambda i: (0, i, 0)),
            pl.BlockSpec((_BLK, HEADS), lambda i: (i, 0)),
            pl.BlockSpec((_BLK, HEADS), lambda i: (i, 0)),
        ],
        out_shape=[
            jax.ShapeDtypeStruct((NU, HEADS * HID), _F32),
            jax.ShapeDtypeStruct((NU, HEADS), _F32),
            jax.ShapeDtypeStruct((HEADS, NU, HID), _F32),
            jax.ShapeDtypeStruct((NU, HEADS), _F32),
            jax.ShapeDtypeStruct((NU, HEADS), _F32),
        ],
        name="tc_pre_user",
    )(user_x, p['user_proj_w'], p['user_proj_b'], p['rel_Wdst'], p['rel_a'],
      p['soc_w'], p['soc_as'], p['soc_ad'])

    padn = lambda x: jnp.pad(x, ((0, 0), (0, NR - NU)))
    socT = socT3.reshape(HEADS * NU, HID)
    sdU = sdU.T
    socS = socS.T
    socD = socD.T

    rel6, = _rel_edge_pass(padn(sdU), ssF, wedge, u_e, f_e, om_e, z_e, zsN)
    rel6 = rel6.reshape(12, NR)
    Ssoc, ACCsoc = _wgs_heads(socT, padn(socS), padn(socD), s_src, s_dst,
                              zsN, zsA)
    Ssoc = Ssoc.reshape(2, NR)
    ACCsoc = ACCsoc.reshape(2, NR, HID)
    sda, t0a, t255a = _dsl_pass(u_e, f_e, om_e)
    sda = sda[:NU]
    t0a = t0a[:NU]
    t255a = t255a[:NU]

    rel_out, = _tc_call(
        _tc_rel, [(NU, HID)],
        hdU, rel6, p['rel_Wz'], p['rel_bias'], name="tc_rel")

    user, h1, s1S, s1D = _tc_call(
        _tc_mid,
        [(NU, HID), (NU, HID), (1, NU), (1, NU)],
        rel_out, socT, padn(socS), padn(socD), Ssoc, ACCsoc,
        p['soc_b'], p['g1_w'], p['g1_as'], p['g1_ad'],
        name="tc_mid")

    Sg1, ACC1 = _wgs_split(h1, padn(s1S), padn(s1D), s_src, s_dst, zsN, zsA)
    Sg1 = Sg1.reshape(2, NR)
    ACC1 = ACC1.reshape(2, NR, HID)

    h2, s2S, s2D = _tc_call(
        _tc_g1fin,
        [(NU, HID), (1, NU), (1, NU)],
        h1, s1S, s1D, Sg1, ACC1, p['g1_b'], p['g2_w'], p['g2_as'], p['g2_ad'],
        name="tc_g1fin")

    Sg2, ACC2 = _wgs_split(h2, padn(s2S), padn(s2D), s_src, s_dst, zsN, zsA)
    Sg2 = Sg2.reshape(2, NR)
    ACC2 = ACC2.reshape(2, NR, HID)

    g2d, = _tc_call(
        _tc_att, [(1, HID)],
        user, h2, s2S, s2D, Sg2, ACC2,
        p['g2_b'], p['Wq'], p['bq'], p['Wk'], p['bk'], p['Wv'], p['bv'],
        p['Wne'], p['bne'],
        name="tc_att")

    o, d = _tc_call(
        _tc_out, [(NU, HID), (NU, HID)],
        user, g2d, sda, t0a, t255a,
        p['Wme'], p['bme'],
        p['conv_w'][:, 0, :], p['conv_b'], p['dsl_w'], p['dsl_b'],
        p['op1_w'], p['op1_b'], p['op2_w'], p['op2_b'],
        name="tc_out")

    return o, g2d[0], d


# speedup vs baseline: 63.7450x; 1.3179x over previous
import functools

import jax
import jax.numpy as jnp
from jax import lax
from jax.experimental import pallas as pl
from jax.experimental.pallas import tpu as pltpu
from jax.experimental.pallas import tpu_sc as plsc

NU = 10000
NF = 256
HID = 64
HEADS = 2
NR = 10112
PAD_E = 163840
K = 1024
KD = 8192
NTILE = 16
NW = 2 * NTILE
RPT = NR // NTILE
DSL_R = 320
_F32 = jnp.float32
_I32 = jnp.int32

_MESH = dict(core_axis_name="c", subcore_axis_name="s")


def _leaky(x):
    return jnp.where(x >= 0, x, x * jnp.float32(0.2))


@functools.lru_cache(maxsize=None)
def _make_wgs(nt, edge_split):
    epc = PAD_E // 2 if edge_split else PAD_E
    per_tile = epc // NTILE
    nch = per_tile // K

    @functools.partial(
        pl.kernel,
        out_type=[jax.ShapeDtypeStruct((2 * NR,), _F32),
                  jax.ShapeDtypeStruct((2 * NR, HID), _F32)],
        mesh=plsc.VectorSubcoreMesh(**_MESH),
        compiler_params=pltpu.CompilerParams(needs_layout_passes=False, use_tc_tiling_on_sc=False),
        scratch_types=[
            pltpu.VMEM((NR,), _F32),
            pltpu.VMEM((NR,), _F32),
            pltpu.VMEM((K,), _I32),
            pltpu.VMEM((K,), _I32),
            pltpu.VMEM((K,), _I32),
            pltpu.VMEM((K,), _F32),
            pltpu.VMEM((K, HID), _F32),
            pltpu.VMEM_SHARED((NR,), _F32),
            pltpu.VMEM_SHARED((NR, HID), _F32),
            pltpu.SemaphoreType.DMA,
            pltpu.SemaphoreType.DMA,
        ],
    )
    def wgs(table_hbm, sS_hbm, sD_hbm, src_hbm, dst_hbm, zs_hbm, za_hbm,
            S_out, ACC_out,
            sS_loc, sD_loc, src_buf, dst_buf, gidx, w_buf, rows, s_sh, acc_sh,
            sem, sem2):
        cid = lax.axis_index("c")
        tid = lax.axis_index("s")
        tsel = cid if nt == 2 else 0

        @pl.when(tid == 0)
        def _zero():
            pltpu.sync_copy(zs_hbm, s_sh)
            pltpu.sync_copy(za_hbm, acc_sh)

        pltpu.sync_copy(sS_hbm.at[tsel], sS_loc)
        pltpu.sync_copy(sD_hbm.at[tsel], sD_loc)
        plsc.subcore_barrier()

        ebase = (cid * epc if edge_split else 0) + tid * per_tile
        toff = tsel * NU

        def chunk(ch, carry):
            off = ebase + ch * K
            pltpu.sync_copy(src_hbm.at[pl.ds(off, K)], src_buf)
            pltpu.sync_copy(dst_hbm.at[pl.ds(off, K)], dst_buf)

            def ofs(j, c2):
                gidx[pl.ds(j * 16, 16)] = src_buf[pl.ds(j * 16, 16)] + toff
                return c2

            lax.fori_loop(0, K // 16, ofs, 0)
            gdma = pltpu.async_copy(table_hbm.at[gidx], rows, sem)

            def grp(j, c2):
                sv = src_buf[pl.ds(j * 16, 16)]
                dv = dst_buf[pl.ds(j * 16, 16)]
                ss = plsc.load_gather(sS_loc, [sv])
                sd = plsc.load_gather(sD_loc, [dv])
                w_buf[pl.ds(j * 16, 16)] = jnp.exp(_leaky(ss + sd))
                return c2

            lax.fori_loop(0, K // 16, grp, 0)
            wdma = pltpu.async_copy(w_buf, s_sh.at[dst_buf], sem2, add=True)
            gdma.wait()

            def edge(e, c2):
                lanes = jnp.full((16,), 0, _I32) + e
                wsp = plsc.load_gather(w_buf, [lanes])
                for q in range(HID // 16):
                    rows[e, pl.ds(q * 16, 16)] = rows[e, pl.ds(q * 16, 16)] * wsp
                return c2

            lax.fori_loop(0, K, edge, 0)
            pltpu.sync_copy(rows, acc_sh.at[dst_buf], add=True)
            wdma.wait()
            return carry

        lax.fori_loop(0, nch, chunk, 0)
        plsc.subcore_barrier()
        r0 = tid * RPT
        pltpu.sync_copy(s_sh.at[pl.ds(r0, RPT)],
                        S_out.at[pl.ds(cid * NR + r0, RPT)])
        pltpu.sync_copy(acc_sh.at[pl.ds(r0, RPT)],
                        ACC_out.at[pl.ds(cid * NR + r0, RPT)])

    return wgs


@functools.lru_cache(maxsize=None)
def _make_rel():
    @functools.partial(
        pl.kernel,
        out_type=[jax.ShapeDtypeStruct((12 * NR,), _F32)],
        mesh=plsc.VectorSubcoreMesh(**_MESH),
        compiler_params=pltpu.CompilerParams(needs_layout_passes=False, use_tc_tiling_on_sc=False),
        scratch_types=[
            pltpu.VMEM((NR,), _F32),
            pltpu.VMEM((NR,), _F32),
            pltpu.VMEM((NF + 16,), _F32),
            pltpu.VMEM((NF + 16,), _F32),
            pltpu.VMEM((64,), _F32),
            pltpu.VMEM((K,), _I32),
            pltpu.VMEM((K,), _I32),
            pltpu.VMEM((K,), _F32),
            pltpu.VMEM((K,), _F32),
            [pltpu.VMEM((K,), _F32) for _ in range(6)],
            [pltpu.VMEM_SHARED((NR,), _F32) for _ in range(6)],
        ],
    )
    def rel(sDu_hbm, sSf_hbm, wedge_hbm, u_hbm, f_hbm, om_hbm, z_hbm,
            zs_hbm, out,
            sD0, sD1, sS0, sS1, wg, u_buf, f_buf, om_buf, z_buf, qb, qsh):
        cid = lax.axis_index("c")
        tid = lax.axis_index("s")

        @pl.when(tid == 0)
        def _zero():
            for q in range(6):
                pltpu.sync_copy(zs_hbm, qsh[q])

        pltpu.sync_copy(sDu_hbm.at[0], sD0)
        pltpu.sync_copy(sDu_hbm.at[1], sD1)
        pltpu.sync_copy(sSf_hbm.at[0], sS0)
        pltpu.sync_copy(sSf_hbm.at[1], sS1)
        pltpu.sync_copy(wedge_hbm, wg)
        plsc.subcore_barrier()

        per_tile = (PAD_E // 2) // NTILE
        ebase = cid * (PAD_E // 2) + tid * per_tile
        we00 = wg[pl.ds(0, 16)]
        we01 = wg[pl.ds(16, 16)]
        we10 = wg[pl.ds(32, 16)]
        we11 = wg[pl.ds(48, 16)]
        one = jnp.full((16,), 1.0, _F32)

        def chunk(ch, carry):
            off = ebase + ch * K
            pltpu.sync_copy(u_hbm.at[pl.ds(off, K)], u_buf)
            pltpu.sync_copy(f_hbm.at[pl.ds(off, K)], f_buf)
            pltpu.sync_copy(om_hbm.at[pl.ds(off, K)], om_buf)
            pltpu.sync_copy(z_hbm.at[pl.ds(off, K)], z_buf)

            def grp(j, c2):
                sl = pl.ds(j * 16, 16)
                uv = u_buf[sl]
                fv = f_buf[sl]
                om = om_buf[sl]
                zv = z_buf[sl]
                w0 = jnp.exp(_leaky(plsc.load_gather(sD0, [uv])
                                    + plsc.load_gather(sS0, [fv])))
                w1 = jnp.exp(_leaky(plsc.load_gather(sD1, [uv])
                                    + plsc.load_gather(sS1, [fv])))
                ew0 = one / (one + jnp.exp(-(om * we00 + zv * we01)))
                ew1 = one / (one + jnp.exp(-(om * we10 + zv * we11)))
                c0 = w0 * ew0 * om
                c1 = w1 * ew1 * om
                qb[0][sl] = w0
                qb[1][sl] = w1
                qb[2][sl] = c0
                qb[3][sl] = c1
                qb[4][sl] = c0 * zv
                qb[5][sl] = c1 * zv
                return c2

            lax.fori_loop(0, K // 16, grp, 0)
            for q in range(6):
                pltpu.sync_copy(qb[q], qsh[q].at[u_buf], add=True)
            return carry

        lax.fori_loop(0, per_tile // K, chunk, 0)
        plsc.subcore_barrier()
        r0 = tid * RPT
        for q in range(6):
            pltpu.sync_copy(qsh[q].at[pl.ds(r0, RPT)],
                            out.at[pl.ds((cid * 6 + q) * NR + r0, RPT)])

    return rel


@functools.lru_cache(maxsize=None)
def _make_dsl():
    @functools.partial(
        pl.kernel,
        out_type=[jax.ShapeDtypeStruct((NW * DSL_R,), _F32),
                  jax.ShapeDtypeStruct((NW * DSL_R,), _F32),
                  jax.ShapeDtypeStruct((NW * DSL_R,), _F32)],
        mesh=plsc.VectorSubcoreMesh(**_MESH),
        compiler_params=pltpu.CompilerParams(needs_layout_passes=False, use_tc_tiling_on_sc=False),
        scratch_types=[
            pltpu.VMEM((DSL_R * NF,), _F32),
            pltpu.VMEM((KD,), _I32),
            pltpu.VMEM((KD,), _F32),
            pltpu.VMEM((DSL_R, 16), _F32),
            pltpu.VMEM((DSL_R,), _F32),
            pltpu.VMEM((DSL_R,), _F32),
            pltpu.VMEM((DSL_R,), _F32),
        ],
    )
    def dsl(key_hbm, om_hbm, sd_out, t0_out, t255_out,
            shard, key_buf, om_buf, spart, srow, t0_loc, t255_loc):
        cid = lax.axis_index("c")
        tid = lax.axis_index("s")
        w = cid * NTILE + tid
        base = w * DSL_R
        zeros16 = jnp.zeros((16,), _F32)

        def z(i, c):
            shard[pl.ds(i * 16, 16)] = zeros16
            return c

        lax.fori_loop(0, DSL_R * NF // 16, z, 0)

        kbase = base * NF

        def chunk(ch, carry):
            off = ch * KD
            pltpu.sync_copy(key_hbm.at[pl.ds(off, KD)], key_buf)
            pltpu.sync_copy(om_hbm.at[pl.ds(off, KD)], om_buf)

            def grp(j, c2):
                sl = pl.ds(j * 16, 16)
                kv = key_buf[sl] - kbase
                m = jnp.logical_and(kv >= 0, kv < DSL_R * NF)
                lidx = jnp.where(m, kv, 0)
                val = jnp.full((16,), 1.0, _F32) - om_buf[sl]
                plsc.store_scatter(shard, [lidx], val, mask=m)
                return c2

            lax.fori_loop(0, KD // 16, grp, 0)
            return carry

        lax.fori_loop(0, PAD_E // KD, chunk, 0)

        def rowsum(r, carry):
            def qs(q, acc):
                return acc + shard[pl.ds(r * NF + q * 16, 16)]

            spart[r, pl.ds(0, 16)] = lax.fori_loop(0, NF // 16, qs, zeros16)
            return carry

        lax.fori_loop(0, DSL_R, rowsum, 0)

        iota = jnp.arange(16, dtype=_I32)

        def rowred(rg, carry):
            ridx = iota + rg * 16
            tot = jnp.zeros((16,), _F32)
            for c in range(16):
                cidx = jnp.full((16,), c, _I32)
                tot = tot + plsc.load_gather(spart, [ridx, cidx])
            srow[pl.ds(rg * 16, 16)] = tot
            return carry

        lax.fori_loop(0, DSL_R // 16, rowred, 0)

        def cols(rg, carry):
            ridx = (iota + rg * 16) * NF
            t0_loc[pl.ds(rg * 16, 16)] = plsc.load_gather(shard, [ridx])
            t255_loc[pl.ds(rg * 16, 16)] = plsc.load_gather(
                shard, [ridx + (NF - 1)])
            return carry

        lax.fori_loop(0, DSL_R // 16, cols, 0)
        pltpu.sync_copy(srow, sd_out.at[pl.ds(base, DSL_R)])
        pltpu.sync_copy(t0_loc, t0_out.at[pl.ds(base, DSL_R)])
        pltpu.sync_copy(t255_loc, t255_out.at[pl.ds(base, DSL_R)])

    return dsl


def _rel_edge_pass(*args):
    return _make_rel()(*args)


def _wgs_heads(*args):
    return _make_wgs(2, False)(*args)


def _wgs_split(*args):
    return _make_wgs(1, True)(*args)


def _dsl_pass(*args):
    return _make_dsl()(*args)


_BLK = 1000


def _tc_pre_feat(feat_embed, p_Wf, p_bf, p_Wsrc, p_a, o_ssF):
    xf = feat_embed[...] @ p_Wf[...].T + p_bf[...][None, :]
    hsF = xf @ p_Wsrc[...].T
    a = p_a[...][0]
    hsF3 = hsF.reshape(NF, HEADS, HID)
    for h in range(HEADS):
        o_ssF[h, :] = hsF3[:, h, :] @ a[h, HID:]


def _tc_pre_user(user_x, p_Wu, p_bu, p_Wdst, p_a, p_Wsoc, p_as, p_ad,
                 o_hdU, o_sdU, o_socT, o_socS, o_socD):
    xu = user_x[...] @ p_Wu[...].T + p_bu[...][None, :]
    hdU = xu @ p_Wdst[...].T
    o_hdU[...] = hdU
    a = p_a[...][0]
    hdU3 = hdU.reshape(_BLK, HEADS, HID)
    for h in range(HEADS):
        o_sdU[:, h] = hdU3[:, h, :] @ a[h, :HID]
    hsoc = xu @ p_Wsoc[...].T
    hsoc3 = hsoc.reshape(_BLK, HEADS, HID)
    for h in range(HEADS):
        o_socT[h, :, :] = hsoc3[:, h, :]
        o_socS[:, h] = hsoc3[:, h, :] @ p_as[...][0, h]
        o_socD[:, h] = hsoc3[:, h, :] @ p_ad[...][0, h]


def _tc_rel(hdU, rel6, p_Wz, p_rb, o_rel):
    r = rel6[...]
    hdUv = hdU[...]
    Wzv = p_Wz[...][:, 0]
    rel_out = jnp.zeros((NU, HID), _F32)
    for h in range(HEADS):
        S = r[h, :NU] + r[6 + h, :NU]
        A = r[2 + h, :NU] + r[8 + h, :NU]
        B = r[4 + h, :NU] + r[10 + h, :NU]
        inv = 1.0 / (S + 1e-16)
        rel_out = rel_out + (A * inv)[:, None] * hdUv[:, h * HID:(h + 1) * HID] \
            + (B * inv)[:, None] * Wzv[None, :]
    o_rel[...] = rel_out / HEADS + p_rb[...][None, :]


def _tc_mid(rel_out, socT, socS, socD, Ssoc, ACCsoc,
            p_sb, p_g1w, p_g1as, p_g1ad,
            o_user, o_h1, o_s1S, o_s1D):
    Ss = Ssoc[...]
    ACC = ACCsoc[...]
    soc_out = jnp.zeros((NU, HID), _F32)
    for h in range(HEADS):
        hsoc = socT[...][h * NU:(h + 1) * NU, :]
        wself = jnp.exp(_leaky(socS[...][h, :NU] + socD[...][h, :NU]))
        S = Ss[h, :NU] + wself
        AC = ACC[h, :NU, :] + wself[:, None] * hsoc
        soc_out = soc_out + AC / (S + 1e-16)[:, None]
    soc_out = soc_out / HEADS + p_sb[...][None, :]

    user = jnp.maximum(rel_out[...] + soc_out, 0.0)
    o_user[...] = user
    h1 = user @ p_g1w[...].T
    o_h1[...] = h1
    o_s1S[0, :] = h1 @ p_g1as[...][0, 0]
    o_s1D[0, :] = h1 @ p_g1ad[...][0, 0]


def _tc_g1fin(h1, s1S, s1D, Sg, ACCg, p_g1b, p_g2w, p_g2as, p_g2ad,
              o_h2, o_s2S, o_s2D):
    S2 = Sg[...]
    A2 = ACCg[...]
    wself = jnp.exp(_leaky(s1S[...][0, :] + s1D[...][0, :]))
    S = S2[0, :NU] + S2[1, :NU] + wself
    AC = A2[0, :NU, :] + A2[1, :NU, :] + wself[:, None] * h1[...]
    hr = jnp.maximum(AC / (S + 1e-16)[:, None] + p_g1b[...][None, :], 0.0)
    h2 = hr @ p_g2w[...].T
    o_h2[...] = h2
    o_s2S[0, :] = h2 @ p_g2as[...][0, 0]
    o_s2D[0, :] = h2 @ p_g2ad[...][0, 0]


def _tc_att(user, h2, s2S, s2D, Sg, ACCg,
            p_g2b, p_Wq, p_bq, p_Wk, p_bk, p_Wv, p_bv, p_Wne, p_bne,
            o_g):
    S2 = Sg[...]
    A2 = ACCg[...]
    wself = jnp.exp(_leaky(s2S[...][0, :] + s2D[...][0, :]))
    S = S2[0, :NU] + S2[1, :NU] + wself
    AC = A2[0, :NU, :] + A2[1, :NU, :] + wself[:, None] * h2[...]
    neighbor_h = AC / (S + 1e-16)[:, None] + p_g2b[...][None, :]

    u = user[...]
    Q = u @ p_Wq[...].T + p_bq[...][None, :]
    Km = neighbor_h @ p_Wk[...].T + p_bk[...][None, :]
    V = neighbor_h @ p_Wv[...].T + p_bv[...][None, :]
    scores = (Q * Km).sum(-1) / jnp.sqrt(jnp.float32(HID))
    wat = jax.nn.softmax(scores, axis=0)
    gv = wat @ V
    o_g[0, :] = gv @ p_Wne[...].T + p_bne[...]


def _tc_out(user, gin, sda, t0a, t255a,
            p_Wme, p_bme, p_cw, p_cb, p_dw, p_db, p_o1w, p_o1b,
            p_o2w, p_o2b,
            o_o, o_d):
    g = gin[...][0]
    user_h = user[...] @ p_Wme[...].T + p_bme[...][None, :] + g[None, :]

    Sd = sda[...]
    t0 = t0a[...]
    t255 = t255a[...]
    cw = p_cw[...]
    dw = p_dw[...]
    v1 = dw @ cw.sum(1) / NF
    v2 = dw @ cw[:, 0] / NF
    v3 = dw @ cw[:, 2] / NF
    v0 = dw @ p_cb[...] + p_db[...]
    d = (Sd[:, None] * v1[None, :] - t255[:, None] * v2[None, :]
         - t0[:, None] * v3[None, :] + v0[None, :])
    o_d[...] = d

    W1 = p_o1w[...]
    gterm = g @ W1[:, HID:2 * HID].T + p_o1b[...]
    o1 = jnp.maximum(user_h @ W1[:, :HID].T + d @ W1[:, 2 * HID:].T
                     + gterm[None, :], 0.0)
    o_o[...] = o1 @ p_o2w[...].T + p_o2b[...][None, :]


def _tc_key(u, f, o_key):
    o_key[...] = u[...] * NF + f[...]


def _pallas(body, **kw):
    return pl.pallas_call(body, **kw)


def _tc_call(body, out_shapes, *args, name):
    return _pallas(
        body,
        out_shape=[jax.ShapeDtypeStruct(s, _F32) for s in out_shapes],
        name=name,
    )(*args)


def kernel(user_x, feat_embed, has_edge_attr, params, has_edge_index,
           social_edge_index):
    p = params
    E = has_edge_index.shape[1]
    npad = PAD_E - E
    spread = (jnp.arange(npad, dtype=_I32) % NU).astype(_I32)
    sent16 = (jnp.arange(npad, dtype=_I32) % 16).astype(_I32)
    zf = jnp.zeros((npad,), _F32)

    u_e = jnp.concatenate([has_edge_index[0].astype(_I32), spread])
    f_e = jnp.concatenate([has_edge_index[1].astype(_I32), NF + sent16])
    om_e = jnp.concatenate([has_edge_attr[:, 0], zf])
    z_e = jnp.concatenate([has_edge_attr[:, 1], zf])
    s_src = jnp.concatenate([social_edge_index[0].astype(_I32), NU + sent16])
    s_dst = jnp.concatenate([social_edge_index[1].astype(_I32), spread])

    zsN = jnp.zeros((NR,), _F32)
    zsA = jnp.zeros((NR, HID), _F32)
    wedge = jnp.repeat(p['rel_Wedge'].reshape(4), 16)

    ssF, = _tc_call(
        _tc_pre_feat, [(HEADS, NF)],
        feat_embed, p['feat_proj_w'], p['feat_proj_b'], p['rel_Wsrc'],
        p['rel_a'], name="tc_pre_feat")

    full = lambda shp: pl.BlockSpec(shp, lambda i: tuple(0 for _ in shp))
    hdU, sdU, socT3, socS, socD = _pallas(
        _tc_pre_user,
        grid=(NU // _BLK,),
        in_specs=[
            pl.BlockSpec((_BLK, 128), lambda i: (i, 0)),
            full((HID, 128)), full((HID,)), full((HEADS * HID, HID)),
            full((1, HEADS, 2 * HID)), full((HEADS * HID, HID)),
            full((1, HEADS, HID)), full((1, HEADS, HID)),
        ],
        out_specs=[
            pl.BlockSpec((_BLK, HEADS * HID), lambda i: (i, 0)),
            pl.BlockSpec((_BLK, HEADS), lambda i: (i, 0)),
            pl.BlockSpec((HEADS, _BLK, HID), lambda i: (0, i, 0)),
            pl.BlockSpec((_BLK, HEADS), lambda i: (i, 0)),
            pl.BlockSpec((_BLK, HEADS), lambda i: (i, 0)),
        ],
        out_shape=[
            jax.ShapeDtypeStruct((NU, HEADS * HID), _F32),
            jax.ShapeDtypeStruct((NU, HEADS), _F32),
            jax.ShapeDtypeStruct((HEADS, NU, HID), _F32),
            jax.ShapeDtypeStruct((NU, HEADS), _F32),
            jax.ShapeDtypeStruct((NU, HEADS), _F32),
        ],
        name="tc_pre_user",
    )(user_x, p['user_proj_w'], p['user_proj_b'], p['rel_Wdst'], p['rel_a'],
      p['soc_w'], p['soc_as'], p['soc_ad'])

    padn = lambda x: jnp.pad(x, ((0, 0), (0, NR - NU)))
    neg = jnp.float32(-1e30)

    def padsrc(x):
        n = x.shape[0]
        return jnp.concatenate(
            [x, jnp.full((n, 16), neg),
             jnp.zeros((n, NR - NU - 16), _F32)], axis=1)

    socT = jnp.pad(socT3.reshape(HEADS * NU, HID), ((0, 16), (0, 0)))
    sdU = sdU.T
    socS = socS.T
    socD = socD.T
    ssF = jnp.concatenate([ssF, jnp.full((HEADS, 16), neg)], axis=1)

    rel6, = _rel_edge_pass(padn(sdU), ssF, wedge, u_e, f_e, om_e, z_e, zsN)
    rel6 = rel6.reshape(12, NR)
    Ssoc, ACCsoc = _wgs_heads(socT, padsrc(socS), padn(socD), s_src, s_dst,
                              zsN, zsA)
    Ssoc = Ssoc.reshape(2, NR)
    ACCsoc = ACCsoc.reshape(2, NR, HID)
    key_raw, = _pallas(
        _tc_key,
        out_shape=[jax.ShapeDtypeStruct((E,), _I32)],
        name="tc_key")(has_edge_index[0].astype(_I32),
                       has_edge_index[1].astype(_I32))
    key_e = jnp.concatenate(
        [key_raw, jnp.full((npad,), NW * DSL_R * NF, _I32)])
    sda, t0a, t255a = _dsl_pass(key_e, om_e)
    sda = sda[:NU]
    t0a = t0a[:NU]
    t255a = t255a[:NU]

    rel_out, = _tc_call(
        _tc_rel, [(NU, HID)],
        hdU, rel6, p['rel_Wz'], p['rel_bias'], name="tc_rel")

    user, h1, s1S, s1D = _tc_call(
        _tc_mid,
        [(NU, HID), (NU, HID), (1, NU), (1, NU)],
        rel_out, socT, padn(socS), padn(socD), Ssoc, ACCsoc,
        p['soc_b'], p['g1_w'], p['g1_as'], p['g1_ad'],
        name="tc_mid")

    Sg1, ACC1 = _wgs_split(jnp.pad(h1, ((0, 16), (0, 0))), padsrc(s1S),
                           padn(s1D), s_src, s_dst, zsN, zsA)
    Sg1 = Sg1.reshape(2, NR)
    ACC1 = ACC1.reshape(2, NR, HID)

    h2, s2S, s2D = _tc_call(
        _tc_g1fin,
        [(NU, HID), (1, NU), (1, NU)],
        h1, s1S, s1D, Sg1, ACC1, p['g1_b'], p['g2_w'], p['g2_as'], p['g2_ad'],
        name="tc_g1fin")

    Sg2, ACC2 = _wgs_split(jnp.pad(h2, ((0, 16), (0, 0))), padsrc(s2S),
                           padn(s2D), s_src, s_dst, zsN, zsA)
    Sg2 = Sg2.reshape(2, NR)
    ACC2 = ACC2.reshape(2, NR, HID)

    g2d, = _tc_call(
        _tc_att, [(1, HID)],
        user, h2, s2S, s2D, Sg2, ACC2,
        p['g2_b'], p['Wq'], p['bq'], p['Wk'], p['bk'], p['Wv'], p['bv'],
        p['Wne'], p['bne'],
        name="tc_att")

    o, d = _tc_call(
        _tc_out, [(NU, HID), (NU, HID)],
        user, g2d, sda, t0a, t255a,
        p['Wme'], p['bme'],
        p['conv_w'][:, 0, :], p['conv_b'], p['dsl_w'], p['dsl_b'],
        p['op1_w'], p['op1_b'], p['op2_w'], p['op2_b'],
        name="tc_out")

    return o, g2d[0], d


# speedup vs baseline: 65.8611x; 1.0332x over previous
import functools

import jax
import jax.numpy as jnp
from jax import lax
from jax.experimental import pallas as pl
from jax.experimental.pallas import tpu as pltpu
from jax.experimental.pallas import tpu_sc as plsc

NU = 10000
NF = 256
HID = 64
HEADS = 2
NR = 10112
PAD_E = 163840
K = 1024
KD = 8192
NTILE = 16
NW = 2 * NTILE
RPT = NR // NTILE
DSL_R = 320
_F32 = jnp.float32
_I32 = jnp.int32

_MESH = dict(core_axis_name="c", subcore_axis_name="s")


def _leaky(x):
    return jnp.where(x >= 0, x, x * jnp.float32(0.2))


@functools.lru_cache(maxsize=None)
def _make_wgs(nt, edge_split):
    epc = PAD_E // 2 if edge_split else PAD_E
    per_tile = epc // NTILE
    nch = per_tile // K

    @functools.partial(
        pl.kernel,
        out_type=[jax.ShapeDtypeStruct((2 * NR,), _F32),
                  jax.ShapeDtypeStruct((2 * NR, HID), _F32)],
        mesh=plsc.VectorSubcoreMesh(**_MESH),
        compiler_params=pltpu.CompilerParams(needs_layout_passes=False, use_tc_tiling_on_sc=False),
        scratch_types=[
            pltpu.VMEM((NR,), _F32),
            pltpu.VMEM((NR,), _F32),
            pltpu.VMEM((K,), _I32),
            pltpu.VMEM((K,), _I32),
            pltpu.VMEM((K,), _I32),
            pltpu.VMEM((K,), _F32),
            pltpu.VMEM((K, HID), _F32),
            pltpu.VMEM_SHARED((NR,), _F32),
            pltpu.VMEM_SHARED((NR, HID), _F32),
            pltpu.SemaphoreType.DMA,
            pltpu.SemaphoreType.DMA,
        ],
    )
    def wgs(table_hbm, sS_hbm, sD_hbm, src_hbm, dst_hbm, zs_hbm, za_hbm,
            S_out, ACC_out,
            sS_loc, sD_loc, src_buf, dst_buf, gidx, w_buf, rows, s_sh, acc_sh,
            sem, sem2):
        cid = lax.axis_index("c")
        tid = lax.axis_index("s")
        tsel = cid if nt == 2 else 0

        @pl.when(tid == 0)
        def _zero():
            pltpu.sync_copy(zs_hbm, s_sh)
            pltpu.sync_copy(za_hbm, acc_sh)

        pltpu.sync_copy(sS_hbm.at[tsel], sS_loc)
        pltpu.sync_copy(sD_hbm.at[tsel], sD_loc)
        plsc.subcore_barrier()

        ebase = (cid * epc if edge_split else 0) + tid * per_tile
        toff = tsel * NU

        def chunk(ch, carry):
            off = ebase + ch * K
            pltpu.sync_copy(src_hbm.at[pl.ds(off, K)], src_buf)
            pltpu.sync_copy(dst_hbm.at[pl.ds(off, K)], dst_buf)

            def ofs(j, c2):
                gidx[pl.ds(j * 16, 16)] = src_buf[pl.ds(j * 16, 16)] + toff
                return c2

            lax.fori_loop(0, K // 16, ofs, 0)
            gdma = pltpu.async_copy(table_hbm.at[gidx], rows, sem)

            def grp(j, c2):
                sv = src_buf[pl.ds(j * 16, 16)]
                dv = dst_buf[pl.ds(j * 16, 16)]
                ss = plsc.load_gather(sS_loc, [sv])
                sd = plsc.load_gather(sD_loc, [dv])
                w_buf[pl.ds(j * 16, 16)] = jnp.exp(_leaky(ss + sd))
                return c2

            lax.fori_loop(0, K // 16, grp, 0)
            wdma = pltpu.async_copy(w_buf, s_sh.at[dst_buf], sem2, add=True)
            gdma.wait()

            def edge(e4, c2):
                for u in range(4):
                    e = e4 * 4 + u
                    lanes = jnp.full((16,), 0, _I32) + e
                    wsp = plsc.load_gather(w_buf, [lanes])
                    for q in range(HID // 16):
                        rows[e, pl.ds(q * 16, 16)] = \
                            rows[e, pl.ds(q * 16, 16)] * wsp
                return c2

            lax.fori_loop(0, K // 4, edge, 0)
            pltpu.sync_copy(rows, acc_sh.at[dst_buf], add=True)
            wdma.wait()
            return carry

        lax.fori_loop(0, nch, chunk, 0)
        plsc.subcore_barrier()
        r0 = tid * RPT
        pltpu.sync_copy(s_sh.at[pl.ds(r0, RPT)],
                        S_out.at[pl.ds(cid * NR + r0, RPT)])
        pltpu.sync_copy(acc_sh.at[pl.ds(r0, RPT)],
                        ACC_out.at[pl.ds(cid * NR + r0, RPT)])

    return wgs


@functools.lru_cache(maxsize=None)
def _make_rel():
    @functools.partial(
        pl.kernel,
        out_type=[jax.ShapeDtypeStruct((12 * NR,), _F32)],
        mesh=plsc.VectorSubcoreMesh(**_MESH),
        compiler_params=pltpu.CompilerParams(needs_layout_passes=False, use_tc_tiling_on_sc=False),
        scratch_types=[
            pltpu.VMEM((NR,), _F32),
            pltpu.VMEM((NR,), _F32),
            pltpu.VMEM((NF + 16,), _F32),
            pltpu.VMEM((NF + 16,), _F32),
            pltpu.VMEM((64,), _F32),
            pltpu.VMEM((K,), _I32),
            pltpu.VMEM((K,), _I32),
            pltpu.VMEM((K,), _F32),
            pltpu.VMEM((K,), _F32),
            [pltpu.VMEM((K,), _F32) for _ in range(6)],
            [pltpu.VMEM_SHARED((NR,), _F32) for _ in range(6)],
        ],
    )
    def rel(sDu_hbm, sSf_hbm, wedge_hbm, u_hbm, f_hbm, om_hbm, z_hbm,
            zs_hbm, out,
            sD0, sD1, sS0, sS1, wg, u_buf, f_buf, om_buf, z_buf, qb, qsh):
        cid = lax.axis_index("c")
        tid = lax.axis_index("s")

        @pl.when(tid == 0)
        def _zero():
            for q in range(6):
                pltpu.sync_copy(zs_hbm, qsh[q])

        pltpu.sync_copy(sDu_hbm.at[0], sD0)
        pltpu.sync_copy(sDu_hbm.at[1], sD1)
        pltpu.sync_copy(sSf_hbm.at[0], sS0)
        pltpu.sync_copy(sSf_hbm.at[1], sS1)
        pltpu.sync_copy(wedge_hbm, wg)
        plsc.subcore_barrier()

        per_tile = (PAD_E // 2) // NTILE
        ebase = cid * (PAD_E // 2) + tid * per_tile
        we00 = wg[pl.ds(0, 16)]
        we01 = wg[pl.ds(16, 16)]
        we10 = wg[pl.ds(32, 16)]
        we11 = wg[pl.ds(48, 16)]
        one = jnp.full((16,), 1.0, _F32)

        def chunk(ch, carry):
            off = ebase + ch * K
            pltpu.sync_copy(u_hbm.at[pl.ds(off, K)], u_buf)
            pltpu.sync_copy(f_hbm.at[pl.ds(off, K)], f_buf)
            pltpu.sync_copy(om_hbm.at[pl.ds(off, K)], om_buf)
            pltpu.sync_copy(z_hbm.at[pl.ds(off, K)], z_buf)

            def grp(j, c2):
                sl = pl.ds(j * 16, 16)
                uv = u_buf[sl]
                fv = f_buf[sl]
                om = om_buf[sl]
                zv = z_buf[sl]
                w0 = jnp.exp(_leaky(plsc.load_gather(sD0, [uv])
                                    + plsc.load_gather(sS0, [fv])))
                w1 = jnp.exp(_leaky(plsc.load_gather(sD1, [uv])
                                    + plsc.load_gather(sS1, [fv])))
                ew0 = one / (one + jnp.exp(-(om * we00 + zv * we01)))
                ew1 = one / (one + jnp.exp(-(om * we10 + zv * we11)))
                c0 = w0 * ew0 * om
                c1 = w1 * ew1 * om
                qb[0][sl] = w0
                qb[1][sl] = w1
                qb[2][sl] = c0
                qb[3][sl] = c1
                qb[4][sl] = c0 * zv
                qb[5][sl] = c1 * zv
                return c2

            lax.fori_loop(0, K // 16, grp, 0)
            for q in range(6):
                pltpu.sync_copy(qb[q], qsh[q].at[u_buf], add=True)
            return carry

        lax.fori_loop(0, per_tile // K, chunk, 0)
        plsc.subcore_barrier()
        r0 = tid * RPT
        for q in range(6):
            pltpu.sync_copy(qsh[q].at[pl.ds(r0, RPT)],
                            out.at[pl.ds((cid * 6 + q) * NR + r0, RPT)])

    return rel


@functools.lru_cache(maxsize=None)
def _make_dsl():
    @functools.partial(
        pl.kernel,
        out_type=[jax.ShapeDtypeStruct((NW * DSL_R,), _F32),
                  jax.ShapeDtypeStruct((NW * DSL_R,), _F32),
                  jax.ShapeDtypeStruct((NW * DSL_R,), _F32)],
        mesh=plsc.VectorSubcoreMesh(**_MESH),
        compiler_params=pltpu.CompilerParams(needs_layout_passes=False, use_tc_tiling_on_sc=False),
        scratch_types=[
            pltpu.VMEM((DSL_R * NF,), _F32),
            pltpu.VMEM((KD,), _I32),
            pltpu.VMEM((KD,), _F32),
            pltpu.VMEM((DSL_R, 16), _F32),
            pltpu.VMEM((DSL_R,), _F32),
            pltpu.VMEM((DSL_R,), _F32),
            pltpu.VMEM((DSL_R,), _F32),
        ],
    )
    def dsl(key_hbm, om_hbm, sd_out, t0_out, t255_out,
            shard, key_buf, om_buf, spart, srow, t0_loc, t255_loc):
        cid = lax.axis_index("c")
        tid = lax.axis_index("s")
        w = cid * NTILE + tid
        base = w * DSL_R
        zeros16 = jnp.zeros((16,), _F32)

        def z(i, c):
            shard[pl.ds(i * 16, 16)] = zeros16
            return c

        lax.fori_loop(0, DSL_R * NF // 16, z, 0)

        kbase = base * NF

        def chunk(ch, carry):
            off = ch * KD
            pltpu.sync_copy(key_hbm.at[pl.ds(off, KD)], key_buf)
            pltpu.sync_copy(om_hbm.at[pl.ds(off, KD)], om_buf)

            def grp(j4, c2):
                for u in range(4):
                    sl = pl.ds((j4 * 4 + u) * 16, 16)
                    kv = key_buf[sl] - kbase
                    m = jnp.logical_and(kv >= 0, kv < DSL_R * NF)
                    lidx = jnp.where(m, kv, 0)
                    val = jnp.full((16,), 1.0, _F32) - om_buf[sl]
                    plsc.store_scatter(shard, [lidx], val, mask=m)
                return c2

            lax.fori_loop(0, KD // 64, grp, 0)
            return carry

        lax.fori_loop(0, PAD_E // KD, chunk, 0)

        def rowsum(r, carry):
            def qs(q, acc):
                return acc + shard[pl.ds(r * NF + q * 16, 16)]

            spart[r, pl.ds(0, 16)] = lax.fori_loop(0, NF // 16, qs, zeros16)
            return carry

        lax.fori_loop(0, DSL_R, rowsum, 0)

        iota = jnp.arange(16, dtype=_I32)

        def rowred(rg, carry):
            ridx = iota + rg * 16
            tot = jnp.zeros((16,), _F32)
            for c in range(16):
                cidx = jnp.full((16,), c, _I32)
                tot = tot + plsc.load_gather(spart, [ridx, cidx])
            srow[pl.ds(rg * 16, 16)] = tot
            return carry

        lax.fori_loop(0, DSL_R // 16, rowred, 0)

        def cols(rg, carry):
            ridx = (iota + rg * 16) * NF
            t0_loc[pl.ds(rg * 16, 16)] = plsc.load_gather(shard, [ridx])
            t255_loc[pl.ds(rg * 16, 16)] = plsc.load_gather(
                shard, [ridx + (NF - 1)])
            return carry

        lax.fori_loop(0, DSL_R // 16, cols, 0)
        pltpu.sync_copy(srow, sd_out.at[pl.ds(base, DSL_R)])
        pltpu.sync_copy(t0_loc, t0_out.at[pl.ds(base, DSL_R)])
        pltpu.sync_copy(t255_loc, t255_out.at[pl.ds(base, DSL_R)])

    return dsl


def _rel_edge_pass(*args):
    return _make_rel()(*args)


def _wgs_heads(*args):
    return _make_wgs(2, False)(*args)


def _wgs_split(*args):
    return _make_wgs(1, True)(*args)


def _dsl_pass(*args):
    return _make_dsl()(*args)


_BLK = 1000


def _tc_pre_feat(feat_embed, p_Wf, p_bf, p_Wsrc, p_a, o_ssF):
    xf = feat_embed[...] @ p_Wf[...].T + p_bf[...][None, :]
    hsF = xf @ p_Wsrc[...].T
    a = p_a[...][0]
    hsF3 = hsF.reshape(NF, HEADS, HID)
    for h in range(HEADS):
        o_ssF[h, :] = hsF3[:, h, :] @ a[h, HID:]


def _tc_pre_user(user_x, p_Wu, p_bu, p_Wdst, p_a, p_Wsoc, p_as, p_ad,
                 o_hdU, o_sdU, o_socT, o_socS, o_socD):
    xu = user_x[...] @ p_Wu[...].T + p_bu[...][None, :]
    hdU = xu @ p_Wdst[...].T
    o_hdU[...] = hdU
    a = p_a[...][0]
    hdU3 = hdU.reshape(_BLK, HEADS, HID)
    for h in range(HEADS):
        o_sdU[:, h] = hdU3[:, h, :] @ a[h, :HID]
    hsoc = xu @ p_Wsoc[...].T
    hsoc3 = hsoc.reshape(_BLK, HEADS, HID)
    for h in range(HEADS):
        o_socT[h, :, :] = hsoc3[:, h, :]
        o_socS[:, h] = hsoc3[:, h, :] @ p_as[...][0, h]
        o_socD[:, h] = hsoc3[:, h, :] @ p_ad[...][0, h]


def _tc_rel(hdU, rel6, p_Wz, p_rb, o_rel):
    r = rel6[...]
    hdUv = hdU[...]
    Wzv = p_Wz[...][:, 0]
    rel_out = jnp.zeros((NU, HID), _F32)
    for h in range(HEADS):
        S = r[h, :NU] + r[6 + h, :NU]
        A = r[2 + h, :NU] + r[8 + h, :NU]
        B = r[4 + h, :NU] + r[10 + h, :NU]
        inv = 1.0 / (S + 1e-16)
        rel_out = rel_out + (A * inv)[:, None] * hdUv[:, h * HID:(h + 1) * HID] \
            + (B * inv)[:, None] * Wzv[None, :]
    o_rel[...] = rel_out / HEADS + p_rb[...][None, :]


def _tc_mid(rel_out, socT, socS, socD, Ssoc, ACCsoc,
            p_sb, p_g1w, p_g1as, p_g1ad,
            o_user, o_h1, o_s1S, o_s1D):
    Ss = Ssoc[...]
    ACC = ACCsoc[...]
    soc_out = jnp.zeros((NU, HID), _F32)
    for h in range(HEADS):
        hsoc = socT[...][h * NU:(h + 1) * NU, :]
        wself = jnp.exp(_leaky(socS[...][h, :NU] + socD[...][h, :NU]))
        S = Ss[h, :NU] + wself
        AC = ACC[h, :NU, :] + wself[:, None] * hsoc
        soc_out = soc_out + AC / (S + 1e-16)[:, None]
    soc_out = soc_out / HEADS + p_sb[...][None, :]

    user = jnp.maximum(rel_out[...] + soc_out, 0.0)
    o_user[...] = user
    h1 = user @ p_g1w[...].T
    o_h1[...] = h1
    o_s1S[0, :] = h1 @ p_g1as[...][0, 0]
    o_s1D[0, :] = h1 @ p_g1ad[...][0, 0]


def _tc_g1fin(h1, s1S, s1D, Sg, ACCg, p_g1b, p_g2w, p_g2as, p_g2ad,
              o_h2, o_s2S, o_s2D):
    S2 = Sg[...]
    A2 = ACCg[...]
    wself = jnp.exp(_leaky(s1S[...][0, :] + s1D[...][0, :]))
    S = S2[0, :NU] + S2[1, :NU] + wself
    AC = A2[0, :NU, :] + A2[1, :NU, :] + wself[:, None] * h1[...]
    hr = jnp.maximum(AC / (S + 1e-16)[:, None] + p_g1b[...][None, :], 0.0)
    h2 = hr @ p_g2w[...].T
    o_h2[...] = h2
    o_s2S[0, :] = h2 @ p_g2as[...][0, 0]
    o_s2D[0, :] = h2 @ p_g2ad[...][0, 0]


def _tc_att(user, h2, s2S, s2D, Sg, ACCg,
            p_g2b, p_Wq, p_bq, p_Wk, p_bk, p_Wv, p_bv, p_Wne, p_bne,
            o_g):
    S2 = Sg[...]
    A2 = ACCg[...]
    wself = jnp.exp(_leaky(s2S[...][0, :] + s2D[...][0, :]))
    S = S2[0, :NU] + S2[1, :NU] + wself
    AC = A2[0, :NU, :] + A2[1, :NU, :] + wself[:, None] * h2[...]
    neighbor_h = AC / (S + 1e-16)[:, None] + p_g2b[...][None, :]

    u = user[...]
    Q = u @ p_Wq[...].T + p_bq[...][None, :]
    Km = neighbor_h @ p_Wk[...].T + p_bk[...][None, :]
    V = neighbor_h @ p_Wv[...].T + p_bv[...][None, :]
    scores = (Q * Km).sum(-1) / jnp.sqrt(jnp.float32(HID))
    wat = jax.nn.softmax(scores, axis=0)
    gv = wat @ V
    o_g[0, :] = gv @ p_Wne[...].T + p_bne[...]


def _tc_out(user, gin, sda, t0a, t255a,
            p_Wme, p_bme, p_cw, p_cb, p_dw, p_db, p_o1w, p_o1b,
            p_o2w, p_o2b,
            o_o, o_d):
    g = gin[...][0]
    user_h = user[...] @ p_Wme[...].T + p_bme[...][None, :] + g[None, :]

    Sd = sda[...]
    t0 = t0a[...]
    t255 = t255a[...]
    cw = p_cw[...]
    dw = p_dw[...]
    v1 = dw @ cw.sum(1) / NF
    v2 = dw @ cw[:, 0] / NF
    v3 = dw @ cw[:, 2] / NF
    v0 = dw @ p_cb[...] + p_db[...]
    d = (Sd[:, None] * v1[None, :] - t255[:, None] * v2[None, :]
         - t0[:, None] * v3[None, :] + v0[None, :])
    o_d[...] = d

    W1 = p_o1w[...]
    gterm = g @ W1[:, HID:2 * HID].T + p_o1b[...]
    o1 = jnp.maximum(user_h @ W1[:, :HID].T + d @ W1[:, 2 * HID:].T
                     + gterm[None, :], 0.0)
    o_o[...] = o1 @ p_o2w[...].T + p_o2b[...][None, :]


def _tc_key(u, f, o_key):
    o_key[...] = u[...] * NF + f[...]


def _pallas(body, **kw):
    return pl.pallas_call(body, **kw)


def _tc_call(body, out_shapes, *args, name):
    return _pallas(
        body,
        out_shape=[jax.ShapeDtypeStruct(s, _F32) for s in out_shapes],
        name=name,
    )(*args)


def kernel(user_x, feat_embed, has_edge_attr, params, has_edge_index,
           social_edge_index):
    p = params
    E = has_edge_index.shape[1]
    npad = PAD_E - E
    spread = (jnp.arange(npad, dtype=_I32) % NU).astype(_I32)
    sent16 = (jnp.arange(npad, dtype=_I32) % 16).astype(_I32)
    zf = jnp.zeros((npad,), _F32)

    u_e = jnp.concatenate([has_edge_index[0].astype(_I32), spread])
    f_e = jnp.concatenate([has_edge_index[1].astype(_I32), NF + sent16])
    om_e = jnp.concatenate([has_edge_attr[:, 0], zf])
    z_e = jnp.concatenate([has_edge_attr[:, 1], zf])
    s_src = jnp.concatenate([social_edge_index[0].astype(_I32), NU + sent16])
    s_dst = jnp.concatenate([social_edge_index[1].astype(_I32), spread])

    zsN = jnp.zeros((NR,), _F32)
    zsA = jnp.zeros((NR, HID), _F32)
    wedge = jnp.repeat(p['rel_Wedge'].reshape(4), 16)

    ssF, = _tc_call(
        _tc_pre_feat, [(HEADS, NF)],
        feat_embed, p['feat_proj_w'], p['feat_proj_b'], p['rel_Wsrc'],
        p['rel_a'], name="tc_pre_feat")

    full = lambda shp: pl.BlockSpec(shp, lambda i: tuple(0 for _ in shp))
    hdU, sdU, socT3, socS, socD = _pallas(
        _tc_pre_user,
        grid=(NU // _BLK,),
        in_specs=[
            pl.BlockSpec((_BLK, 128), lambda i: (i, 0)),
            full((HID, 128)), full((HID,)), full((HEADS * HID, HID)),
            full((1, HEADS, 2 * HID)), full((HEADS * HID, HID)),
            full((1, HEADS, HID)), full((1, HEADS, HID)),
        ],
        out_specs=[
            pl.BlockSpec((_BLK, HEADS * HID), lambda i: (i, 0)),
            pl.BlockSpec((_BLK, HEADS), lambda i: (i, 0)),
            pl.BlockSpec((HEADS, _BLK, HID), lambda i: (0, i, 0)),
            pl.BlockSpec((_BLK, HEADS), lambda i: (i, 0)),
            pl.BlockSpec((_BLK, HEADS), lambda i: (i, 0)),
        ],
        out_shape=[
            jax.ShapeDtypeStruct((NU, HEADS * HID), _F32),
            jax.ShapeDtypeStruct((NU, HEADS), _F32),
            jax.ShapeDtypeStruct((HEADS, NU, HID), _F32),
            jax.ShapeDtypeStruct((NU, HEADS), _F32),
            jax.ShapeDtypeStruct((NU, HEADS), _F32),
        ],
        name="tc_pre_user",
    )(user_x, p['user_proj_w'], p['user_proj_b'], p['rel_Wdst'], p['rel_a'],
      p['soc_w'], p['soc_as'], p['soc_ad'])

    padn = lambda x: jnp.pad(x, ((0, 0), (0, NR - NU)))
    neg = jnp.float32(-1e30)

    def padsrc(x):
        n = x.shape[0]
        return jnp.concatenate(
            [x, jnp.full((n, 16), neg),
             jnp.zeros((n, NR - NU - 16), _F32)], axis=1)

    socT = jnp.pad(socT3.reshape(HEADS * NU, HID), ((0, 16), (0, 0)))
    sdU = sdU.T
    socS = socS.T
    socD = socD.T
    ssF = jnp.concatenate([ssF, jnp.full((HEADS, 16), neg)], axis=1)

    rel6, = _rel_edge_pass(padn(sdU), ssF, wedge, u_e, f_e, om_e, z_e, zsN)
    rel6 = rel6.reshape(12, NR)
    Ssoc, ACCsoc = _wgs_heads(socT, padsrc(socS), padn(socD), s_src, s_dst,
                              zsN, zsA)
    Ssoc = Ssoc.reshape(2, NR)
    ACCsoc = ACCsoc.reshape(2, NR, HID)
    key_raw, = _pallas(
        _tc_key,
        out_shape=[jax.ShapeDtypeStruct((E,), _I32)],
        name="tc_key")(has_edge_index[0].astype(_I32),
                       has_edge_index[1].astype(_I32))
    key_e = jnp.concatenate(
        [key_raw, jnp.full((npad,), NW * DSL_R * NF, _I32)])
    sda, t0a, t255a = _dsl_pass(key_e, om_e)
    sda = sda[:NU]
    t0a = t0a[:NU]
    t255a = t255a[:NU]

    rel_out, = _tc_call(
        _tc_rel, [(NU, HID)],
        hdU, rel6, p['rel_Wz'], p['rel_bias'], name="tc_rel")

    user, h1, s1S, s1D = _tc_call(
        _tc_mid,
        [(NU, HID), (NU, HID), (1, NU), (1, NU)],
        rel_out, socT, padn(socS), padn(socD), Ssoc, ACCsoc,
        p['soc_b'], p['g1_w'], p['g1_as'], p['g1_ad'],
        name="tc_mid")

    Sg1, ACC1 = _wgs_split(jnp.pad(h1, ((0, 16), (0, 0))), padsrc(s1S),
                           padn(s1D), s_src, s_dst, zsN, zsA)
    Sg1 = Sg1.reshape(2, NR)
    ACC1 = ACC1.reshape(2, NR, HID)

    h2, s2S, s2D = _tc_call(
        _tc_g1fin,
        [(NU, HID), (1, NU), (1, NU)],
        h1, s1S, s1D, Sg1, ACC1, p['g1_b'], p['g2_w'], p['g2_as'], p['g2_ad'],
        name="tc_g1fin")

    Sg2, ACC2 = _wgs_split(jnp.pad(h2, ((0, 16), (0, 0))), padsrc(s2S),
                           padn(s2D), s_src, s_dst, zsN, zsA)
    Sg2 = Sg2.reshape(2, NR)
    ACC2 = ACC2.reshape(2, NR, HID)

    g2d, = _tc_call(
        _tc_att, [(1, HID)],
        user, h2, s2S, s2D, Sg2, ACC2,
        p['g2_b'], p['Wq'], p['bq'], p['Wk'], p['bk'], p['Wv'], p['bv'],
        p['Wne'], p['bne'],
        name="tc_att")

    o, d = _tc_call(
        _tc_out, [(NU, HID), (NU, HID)],
        user, g2d, sda, t0a, t255a,
        p['Wme'], p['bme'],
        p['conv_w'][:, 0, :], p['conv_b'], p['dsl_w'], p['dsl_b'],
        p['op1_w'], p['op1_b'], p['op2_w'], p['op2_b'],
        name="tc_out")

    return o, g2d[0], d
